# Initial kernel scaffold; baseline (speedup 1.0000x reference)
import jax
import jax.numpy as jnp
from jax.experimental import pallas as pl


def kernel(original_vertices, original_faces, simplified_vertices, simplified_faces, face_probabilities):
    raise NotImplementedError("write your pallas kernel here")


# speedup vs baseline: 1.4246x; 1.4246x over previous
import functools

import jax
import jax.numpy as jnp
from jax import lax
from jax.experimental import pallas as pl
from jax.experimental.pallas import tpu as pltpu
from jax.experimental.pallas import tpu_sc as plsc

N_OV = 10000
N_OF = 20000
N_SV = 600
N_SF = 1000
S = 8

P_OV = 10240
P_OF = 20480
P_Q = 1024
P_SAMP = 8192

NC = 2
NS = 16
NW = NC * NS
LANES = 16

WF_PER = P_OF // NW
QF_PER = P_Q // NW
QR_PER = P_SAMP // NW
WR_PER = P_OV // NW

BIG = 1e30


def _sc_pack(ov_flat, of_flat, sv_flat, sf_flat, a_flat, b_flat, c_flat):
    mesh = plsc.VectorSubcoreMesh(core_axis_name="c", subcore_axis_name="s")

    @functools.partial(
        pl.kernel,
        mesh=mesh,
        compiler_params=pltpu.CompilerParams(needs_layout_passes=False),
        out_type=[
            jax.ShapeDtypeStruct((8, P_OF), jnp.float32),
            jax.ShapeDtypeStruct((8, P_Q), jnp.float32),
            jax.ShapeDtypeStruct((8, P_SAMP), jnp.float32),
            jax.ShapeDtypeStruct((8, P_OV), jnp.float32),
        ],
        scratch_types=[
            pltpu.VMEM((30720,), jnp.float32),
            pltpu.VMEM((3 * WF_PER,), jnp.int32),
            pltpu.VMEM((2048,), jnp.float32),
            pltpu.VMEM((3072,), jnp.int32),
            pltpu.VMEM((QR_PER,), jnp.float32),
            pltpu.VMEM((QR_PER,), jnp.float32),
            pltpu.VMEM((QR_PER,), jnp.float32),
            pltpu.VMEM((8, WF_PER), jnp.float32),
            pltpu.VMEM((8, 128), jnp.float32),
            pltpu.VMEM((8, QR_PER), jnp.float32),
            pltpu.VMEM((8, 128), jnp.float32),
        ],
    )
    def body(ov_h, of_h, sv_h, sf_h, a_h, b_h, c_h,
             wf_h, qf_h, qr_h, wr_h,
             ov_v, of_v, sv_v, sf_v, a_v, b_v, c_v,
             wf_s, qf_s, qr_s, wr_s):
        wid = lax.axis_index("s") * NC + lax.axis_index("c")

        pltpu.sync_copy(ov_h, ov_v)
        pltpu.sync_copy(of_h.at[pl.ds(wid * 3 * WF_PER, 3 * WF_PER)], of_v)
        pltpu.sync_copy(sv_h, sv_v)
        pltpu.sync_copy(sf_h, sf_v)
        pltpu.sync_copy(a_h.at[pl.ds(wid * QR_PER, QR_PER)], a_v)
        pltpu.sync_copy(b_h.at[pl.ds(wid * QR_PER, QR_PER)], b_v)
        pltpu.sync_copy(c_h.at[pl.ds(wid * QR_PER, QR_PER)], c_v)

        iota = lax.iota(jnp.int32, LANES)
        ones = jnp.full((LANES,), 1.0, jnp.float32)
        zeros = jnp.zeros((LANES,), jnp.float32)

        def gat_f(ref, idx):
            return plsc.load_gather(ref, [idx])

        def tri_coords(faces_ref, fidx):
            g0 = plsc.load_gather(faces_ref, [fidx * 3])
            g1 = plsc.load_gather(faces_ref, [fidx * 3 + 1])
            g2 = plsc.load_gather(faces_ref, [fidx * 3 + 2])
            return g0, g1, g2

        def wf_body(i, carry):
            base = i * LANES
            col_l = base + iota
            g0, g1, g2 = tri_coords(of_v, col_l)
            third = jnp.full((LANES,), 1.0 / 3.0, jnp.float32)
            bx = (gat_f(ov_v, g0 * 3) + gat_f(ov_v, g1 * 3) + gat_f(ov_v, g2 * 3)) * third
            by = (gat_f(ov_v, g0 * 3 + 1) + gat_f(ov_v, g1 * 3 + 1) + gat_f(ov_v, g2 * 3 + 1)) * third
            bz = (gat_f(ov_v, g0 * 3 + 2) + gat_f(ov_v, g1 * 3 + 2) + gat_f(ov_v, g2 * 3 + 2)) * third
            col_g = col_l + wid * WF_PER
            nrm = bx * bx + by * by + bz * bz
            nrm = jnp.where(col_g < N_OF, nrm, BIG)
            sl = pl.ds(base, LANES)
            wf_s[0, sl] = -2.0 * bx
            wf_s[1, sl] = -2.0 * by
            wf_s[2, sl] = -2.0 * bz
            wf_s[3, sl] = ones
            wf_s[4, sl] = nrm
            wf_s[5, sl] = zeros
            wf_s[6, sl] = zeros
            wf_s[7, sl] = zeros
            return carry

        lax.fori_loop(0, WF_PER // LANES, wf_body, 0)

        @pl.when(wid < 8)
        def _qt_fwd():
            def qf_body(i, carry):
                base = i * LANES
                col_g = base + iota + wid * 128
                g0, g1, g2 = tri_coords(sf_v, col_g)
                third = jnp.full((LANES,), 1.0 / 3.0, jnp.float32)
                bx = (gat_f(sv_v, g0 * 3) + gat_f(sv_v, g1 * 3) + gat_f(sv_v, g2 * 3)) * third
                by = (gat_f(sv_v, g0 * 3 + 1) + gat_f(sv_v, g1 * 3 + 1) + gat_f(sv_v, g2 * 3 + 1)) * third
                bz = (gat_f(sv_v, g0 * 3 + 2) + gat_f(sv_v, g1 * 3 + 2) + gat_f(sv_v, g2 * 3 + 2)) * third
                sl = pl.ds(base, LANES)
                qf_s[0, sl] = bx
                qf_s[1, sl] = by
                qf_s[2, sl] = bz
                qf_s[3, sl] = bx * bx + by * by + bz * bz
                qf_s[4, sl] = ones
                qf_s[5, sl] = zeros
                qf_s[6, sl] = zeros
                qf_s[7, sl] = zeros
                return carry

            lax.fori_loop(0, 128 // LANES, qf_body, 0)
            pltpu.sync_copy(qf_s, qf_h.at[:, pl.ds(wid * 128, 128)])

        def qr_body(i, carry):
            base = i * LANES
            col_l = base + iota
            col_g = col_l + wid * QR_PER
            f = lax.shift_right_logical(col_g, 3)
            g0, g1, g2 = tri_coords(sf_v, f)
            sl = pl.ds(base, LANES)
            av = a_v[sl]
            bv = b_v[sl]
            cv = c_v[sl]
            sx = av * gat_f(sv_v, g0 * 3) + bv * gat_f(sv_v, g1 * 3) + cv * gat_f(sv_v, g2 * 3)
            sy = av * gat_f(sv_v, g0 * 3 + 1) + bv * gat_f(sv_v, g1 * 3 + 1) + cv * gat_f(sv_v, g2 * 3 + 1)
            sz = av * gat_f(sv_v, g0 * 3 + 2) + bv * gat_f(sv_v, g1 * 3 + 2) + cv * gat_f(sv_v, g2 * 3 + 2)
            qr_s[0, sl] = sx
            qr_s[1, sl] = sy
            qr_s[2, sl] = sz
            qr_s[3, sl] = sx * sx + sy * sy + sz * sz
            qr_s[4, sl] = ones
            qr_s[5, sl] = zeros
            qr_s[6, sl] = zeros
            qr_s[7, sl] = zeros
            return carry

        lax.fori_loop(0, QR_PER // LANES, qr_body, 0)

        for k in range(3):
            chunk = wid + 32 * k

            @pl.when(chunk < 80)
            def _wr_chunk(chunk=chunk):
                def wr_body(i, carry):
                    base = i * LANES
                    col_g = base + iota + chunk * 128
                    vx = gat_f(ov_v, col_g * 3)
                    vy = gat_f(ov_v, col_g * 3 + 1)
                    vz = gat_f(ov_v, col_g * 3 + 2)
                    nrm = vx * vx + vy * vy + vz * vz
                    nrm = jnp.where(col_g < N_OV, nrm, BIG)
                    sl = pl.ds(base, LANES)
                    wr_s[0, sl] = -2.0 * vx
                    wr_s[1, sl] = -2.0 * vy
                    wr_s[2, sl] = -2.0 * vz
                    wr_s[3, sl] = ones
                    wr_s[4, sl] = nrm
                    wr_s[5, sl] = zeros
                    wr_s[6, sl] = zeros
                    wr_s[7, sl] = zeros
                    return carry

                lax.fori_loop(0, 128 // LANES, wr_body, 0)
                pltpu.sync_copy(wr_s, wr_h.at[:, pl.ds(chunk * 128, 128)])

        pltpu.sync_copy(wf_s, wf_h.at[:, pl.ds(wid * WF_PER, WF_PER)])
        pltpu.sync_copy(qr_s, qr_h.at[:, pl.ds(wid * QR_PER, QR_PER)])

    return body(ov_flat, of_flat, sv_flat, sf_flat, a_flat, b_flat, c_flat)


def _fwd_tc(qt, w, p_pad):
    jblk = 2048
    njb = P_OF // jblk

    def body(qt_ref, w_ref, p_ref, fwd_out, sump_out, acc):
        j = pl.program_id(0)
        d = lax.dot_general(qt_ref[...], w_ref[...], (((0,), (0,)), ((), ())),
                            precision=lax.Precision.HIGHEST,
                            preferred_element_type=jnp.float32)
        m = jnp.min(d, axis=1, keepdims=True)
        acc[...] = jnp.where(j == 0, m, jnp.minimum(acc[...], m))

        @pl.when(j == njb - 1)
        def _():
            p = p_ref[...]
            sp = jnp.sum(p)
            fwd_out[...] = jnp.full((1, 1), jnp.sum(p * acc[...]) + 1e-4 * (float(N_SF) - sp), jnp.float32)
            sump_out[...] = jnp.full((1, 1), sp, jnp.float32)

    return pl.pallas_call(
        body,
        grid=(njb,),
        in_specs=[
            pl.BlockSpec((8, P_Q), lambda j: (0, 0)),
            pl.BlockSpec((8, jblk), lambda j: (0, j)),
            pl.BlockSpec((P_Q, 1), lambda j: (0, 0)),
        ],
        out_specs=[
            pl.BlockSpec((1, 1), lambda j: (0, 0)),
            pl.BlockSpec((1, 1), lambda j: (0, 0)),
        ],
        out_shape=[
            jax.ShapeDtypeStruct((1, 1), jnp.float32),
            jax.ShapeDtypeStruct((1, 1), jnp.float32),
        ],
        scratch_shapes=[pltpu.VMEM((P_Q, 1), jnp.float32)],
    )(qt, w, p_pad)


def _rev_tc(qt, w, mask):
    iblk = 1024
    jblk = 2048
    nib = P_SAMP // iblk
    njb = P_OV // jblk

    def body(qt_ref, w_ref, mask_ref, sum_out, max_out, acc, ssum, smax):
        i = pl.program_id(0)
        j = pl.program_id(1)
        d = lax.dot_general(qt_ref[...], w_ref[...], (((0,), (0,)), ((), ())),
                            precision=lax.Precision.HIGHEST,
                            preferred_element_type=jnp.float32)
        m = jnp.min(d, axis=1, keepdims=True)
        acc[...] = jnp.where(j == 0, m, jnp.minimum(acc[...], m))

        @pl.when(j == njb - 1)
        def _():
            mm = mask_ref[...] * acc[...]
            s = jnp.sum(mm)
            mx = jnp.max(mm)
            ssum[0] = jnp.where(i == 0, s, ssum[0] + s)
            smax[0] = jnp.where(i == 0, mx, jnp.maximum(smax[0], mx))
            sum_out[...] = jnp.full((1, 1), ssum[0], jnp.float32)
            max_out[...] = jnp.full((1, 1), smax[0], jnp.float32)

    return pl.pallas_call(
        body,
        grid=(nib, njb),
        in_specs=[
            pl.BlockSpec((8, iblk), lambda i, j: (0, i)),
            pl.BlockSpec((8, jblk), lambda i, j: (0, j)),
            pl.BlockSpec((iblk, 1), lambda i, j: (i, 0)),
        ],
        out_specs=[
            pl.BlockSpec((1, 1), lambda i, j: (0, 0)),
            pl.BlockSpec((1, 1), lambda i, j: (0, 0)),
        ],
        out_shape=[
            jax.ShapeDtypeStruct((1, 1), jnp.float32),
            jax.ShapeDtypeStruct((1, 1), jnp.float32),
        ],
        scratch_shapes=[
            pltpu.VMEM((iblk, 1), jnp.float32),
            pltpu.SMEM((1,), jnp.float32),
            pltpu.SMEM((1,), jnp.float32),
        ],
    )(qt, w, mask)


def kernel(original_vertices, original_faces, simplified_vertices,
           simplified_faces, face_probabilities):
    f32 = jnp.float32
    ov_flat = jnp.pad(original_vertices.reshape(-1).astype(f32), (0, 30720 - 3 * N_OV))
    of_flat = jnp.pad(original_faces.reshape(-1).astype(jnp.int32), (0, 3 * P_OF - 3 * N_OF))
    sv_flat = jnp.pad(simplified_vertices.reshape(-1).astype(f32), (0, 2048 - 3 * N_SV))
    sf_flat = jnp.pad(simplified_faces.reshape(-1).astype(jnp.int32), (0, 3072 - 3 * N_SF))

    kr = jax.random.key(42)
    kr1, kr2 = jax.random.split(kr)
    r1 = jnp.sqrt(jax.random.uniform(kr1, (N_SF, S, 1), dtype=f32))
    r2 = jax.random.uniform(kr2, (N_SF, S, 1), dtype=f32)
    a_flat = jnp.pad((1.0 - r1).reshape(-1), (0, P_SAMP - N_SF * S))
    b_flat = jnp.pad((r1 * (1.0 - r2)).reshape(-1), (0, P_SAMP - N_SF * S))
    c_flat = jnp.pad((r1 * r2).reshape(-1), (0, P_SAMP - N_SF * S))

    w_fwd, qt_fwd, qt_rev, w_rev = _sc_pack(
        ov_flat, of_flat, sv_flat, sf_flat, a_flat, b_flat, c_flat)

    p_pad = jnp.pad(face_probabilities.astype(f32), (0, P_Q - N_SF)).reshape(P_Q, 1)
    mask = (jnp.arange(P_SAMP) < N_SF * S).astype(f32).reshape(P_SAMP, 1)

    fwd_term, sum_p = _fwd_tc(qt_fwd, w_fwd, p_pad)
    rev_sum, rev_max = _rev_tc(qt_rev, w_rev, mask)

    rev_term = 0.1 * sum_p[0, 0] * rev_sum[0, 0] / rev_max[0, 0]
    return fwd_term[0, 0] + rev_term


# speedup vs baseline: 1.4527x; 1.0197x over previous
import functools

import jax
import jax.numpy as jnp
from jax import lax
from jax.experimental import pallas as pl
from jax.experimental.pallas import tpu as pltpu
from jax.experimental.pallas import tpu_sc as plsc

N_OV = 10000
N_OF = 20000
N_SV = 600
N_SF = 1000
S = 8

P_OV = 10240
P_OF = 20480
P_Q = 1024
P_SAMP = 8192

NC = 2
NS = 16
NW = NC * NS
LANES = 16

WF_PER = P_OF // NW
QF_PER = P_Q // NW
QR_PER = P_SAMP // NW
WR_PER = P_OV // NW

BIG = 1e30


def _sc_pack(ov_flat, of_flat, sv_flat, sf_flat, a_flat, b_flat, c_flat):
    mesh = plsc.VectorSubcoreMesh(core_axis_name="c", subcore_axis_name="s")

    @functools.partial(
        pl.kernel,
        mesh=mesh,
        compiler_params=pltpu.CompilerParams(needs_layout_passes=False),
        out_type=[
            jax.ShapeDtypeStruct((8, P_OF), jnp.float32),
            jax.ShapeDtypeStruct((8, P_Q), jnp.float32),
            jax.ShapeDtypeStruct((8, P_SAMP), jnp.float32),
            jax.ShapeDtypeStruct((8, P_OV), jnp.float32),
        ],
        scratch_types=[
            pltpu.VMEM((30720,), jnp.float32),
            pltpu.VMEM((3 * WF_PER,), jnp.int32),
            pltpu.VMEM((2048,), jnp.float32),
            pltpu.VMEM((3072,), jnp.int32),
            pltpu.VMEM((QR_PER,), jnp.float32),
            pltpu.VMEM((QR_PER,), jnp.float32),
            pltpu.VMEM((QR_PER,), jnp.float32),
            pltpu.VMEM((8, WF_PER), jnp.float32),
            pltpu.VMEM((8, 128), jnp.float32),
            pltpu.VMEM((8, QR_PER), jnp.float32),
            pltpu.VMEM((8, 128), jnp.float32),
        ],
    )
    def body(ov_h, of_h, sv_h, sf_h, a_h, b_h, c_h,
             wf_h, qf_h, qr_h, wr_h,
             ov_v, of_v, sv_v, sf_v, a_v, b_v, c_v,
             wf_s, qf_s, qr_s, wr_s):
        wid = lax.axis_index("s") * NC + lax.axis_index("c")

        pltpu.sync_copy(ov_h, ov_v)
        pltpu.sync_copy(of_h.at[pl.ds(wid * 3 * WF_PER, 3 * WF_PER)], of_v)
        pltpu.sync_copy(sv_h, sv_v)
        pltpu.sync_copy(sf_h, sf_v)
        pltpu.sync_copy(a_h.at[pl.ds(wid * QR_PER, QR_PER)], a_v)
        pltpu.sync_copy(b_h.at[pl.ds(wid * QR_PER, QR_PER)], b_v)
        pltpu.sync_copy(c_h.at[pl.ds(wid * QR_PER, QR_PER)], c_v)

        iota = lax.iota(jnp.int32, LANES)
        ones = jnp.full((LANES,), 1.0, jnp.float32)
        zeros = jnp.zeros((LANES,), jnp.float32)

        def gat_f(ref, idx):
            return plsc.load_gather(ref, [idx])

        def tri_coords(faces_ref, fidx):
            g0 = plsc.load_gather(faces_ref, [fidx * 3])
            g1 = plsc.load_gather(faces_ref, [fidx * 3 + 1])
            g2 = plsc.load_gather(faces_ref, [fidx * 3 + 2])
            return g0, g1, g2

        def wf_body(i, carry):
            base = i * LANES
            col_l = base + iota
            g0, g1, g2 = tri_coords(of_v, col_l)
            third = jnp.full((LANES,), 1.0 / 3.0, jnp.float32)
            bx = (gat_f(ov_v, g0 * 3) + gat_f(ov_v, g1 * 3) + gat_f(ov_v, g2 * 3)) * third
            by = (gat_f(ov_v, g0 * 3 + 1) + gat_f(ov_v, g1 * 3 + 1) + gat_f(ov_v, g2 * 3 + 1)) * third
            bz = (gat_f(ov_v, g0 * 3 + 2) + gat_f(ov_v, g1 * 3 + 2) + gat_f(ov_v, g2 * 3 + 2)) * third
            col_g = col_l + wid * WF_PER
            nrm = bx * bx + by * by + bz * bz
            nrm = jnp.where(col_g < N_OF, nrm, BIG)
            sl = pl.ds(base, LANES)
            wf_s[0, sl] = -2.0 * bx
            wf_s[1, sl] = -2.0 * by
            wf_s[2, sl] = -2.0 * bz
            wf_s[3, sl] = ones
            wf_s[4, sl] = nrm
            wf_s[5, sl] = zeros
            wf_s[6, sl] = zeros
            wf_s[7, sl] = zeros
            return carry

        lax.fori_loop(0, WF_PER // LANES, wf_body, 0)

        @pl.when(wid < 8)
        def _qt_fwd():
            def qf_body(i, carry):
                base = i * LANES
                col_g = base + iota + wid * 128
                g0, g1, g2 = tri_coords(sf_v, col_g)
                third = jnp.full((LANES,), 1.0 / 3.0, jnp.float32)
                bx = (gat_f(sv_v, g0 * 3) + gat_f(sv_v, g1 * 3) + gat_f(sv_v, g2 * 3)) * third
                by = (gat_f(sv_v, g0 * 3 + 1) + gat_f(sv_v, g1 * 3 + 1) + gat_f(sv_v, g2 * 3 + 1)) * third
                bz = (gat_f(sv_v, g0 * 3 + 2) + gat_f(sv_v, g1 * 3 + 2) + gat_f(sv_v, g2 * 3 + 2)) * third
                sl = pl.ds(base, LANES)
                qf_s[0, sl] = bx
                qf_s[1, sl] = by
                qf_s[2, sl] = bz
                qf_s[3, sl] = bx * bx + by * by + bz * bz
                qf_s[4, sl] = ones
                qf_s[5, sl] = zeros
                qf_s[6, sl] = zeros
                qf_s[7, sl] = zeros
                return carry

            lax.fori_loop(0, 128 // LANES, qf_body, 0)
            pltpu.sync_copy(qf_s, qf_h.at[:, pl.ds(wid * 128, 128)])

        def qr_body(i, carry):
            base = i * LANES
            col_l = base + iota
            col_g = col_l + wid * QR_PER
            f = lax.shift_right_logical(col_g, 3)
            g0, g1, g2 = tri_coords(sf_v, f)
            sl = pl.ds(base, LANES)
            av = a_v[sl]
            bv = b_v[sl]
            cv = c_v[sl]
            sx = av * gat_f(sv_v, g0 * 3) + bv * gat_f(sv_v, g1 * 3) + cv * gat_f(sv_v, g2 * 3)
            sy = av * gat_f(sv_v, g0 * 3 + 1) + bv * gat_f(sv_v, g1 * 3 + 1) + cv * gat_f(sv_v, g2 * 3 + 1)
            sz = av * gat_f(sv_v, g0 * 3 + 2) + bv * gat_f(sv_v, g1 * 3 + 2) + cv * gat_f(sv_v, g2 * 3 + 2)
            qr_s[0, sl] = sx
            qr_s[1, sl] = sy
            qr_s[2, sl] = sz
            qr_s[3, sl] = sx * sx + sy * sy + sz * sz
            qr_s[4, sl] = ones
            qr_s[5, sl] = zeros
            qr_s[6, sl] = zeros
            qr_s[7, sl] = zeros
            return carry

        lax.fori_loop(0, QR_PER // LANES, qr_body, 0)

        for k in range(3):
            chunk = wid + 32 * k

            @pl.when(chunk < 80)
            def _wr_chunk(chunk=chunk):
                def wr_body(i, carry):
                    base = i * LANES
                    col_g = base + iota + chunk * 128
                    vx = gat_f(ov_v, col_g * 3)
                    vy = gat_f(ov_v, col_g * 3 + 1)
                    vz = gat_f(ov_v, col_g * 3 + 2)
                    nrm = vx * vx + vy * vy + vz * vz
                    nrm = jnp.where(col_g < N_OV, nrm, BIG)
                    sl = pl.ds(base, LANES)
                    wr_s[0, sl] = -2.0 * vx
                    wr_s[1, sl] = -2.0 * vy
                    wr_s[2, sl] = -2.0 * vz
                    wr_s[3, sl] = ones
                    wr_s[4, sl] = nrm
                    wr_s[5, sl] = zeros
                    wr_s[6, sl] = zeros
                    wr_s[7, sl] = zeros
                    return carry

                lax.fori_loop(0, 128 // LANES, wr_body, 0)
                pltpu.sync_copy(wr_s, wr_h.at[:, pl.ds(chunk * 128, 128)])

        pltpu.sync_copy(wf_s, wf_h.at[:, pl.ds(wid * WF_PER, WF_PER)])
        pltpu.sync_copy(qr_s, qr_h.at[:, pl.ds(wid * QR_PER, QR_PER)])

    return body(ov_flat, of_flat, sv_flat, sf_flat, a_flat, b_flat, c_flat)


def _fwd_tc(qt, w, p_pad):
    jblk = 2048
    njb = P_OF // jblk

    def body(qt_ref, w_ref, p_ref, fwd_out, sump_out, acc):
        j = pl.program_id(0)
        d = lax.dot_general(qt_ref[...], w_ref[...], (((0,), (0,)), ((), ())),
                            precision=lax.Precision.HIGHEST,
                            preferred_element_type=jnp.float32)
        m = jnp.min(d, axis=1, keepdims=True)
        acc[...] = jnp.where(j == 0, m, jnp.minimum(acc[...], m))

        @pl.when(j == njb - 1)
        def _():
            p = p_ref[...]
            sp = jnp.sum(p)
            fwd_out[...] = jnp.full((1, 1), jnp.sum(p * acc[...]) + 1e-4 * (float(N_SF) - sp), jnp.float32)
            sump_out[...] = jnp.full((1, 1), sp, jnp.float32)

    return pl.pallas_call(
        body,
        grid=(njb,),
        in_specs=[
            pl.BlockSpec((8, P_Q), lambda j: (0, 0)),
            pl.BlockSpec((8, jblk), lambda j: (0, j)),
            pl.BlockSpec((P_Q, 1), lambda j: (0, 0)),
        ],
        out_specs=[
            pl.BlockSpec((1, 1), lambda j: (0, 0)),
            pl.BlockSpec((1, 1), lambda j: (0, 0)),
        ],
        out_shape=[
            jax.ShapeDtypeStruct((1, 1), jnp.float32),
            jax.ShapeDtypeStruct((1, 1), jnp.float32),
        ],
        scratch_shapes=[pltpu.VMEM((P_Q, 1), jnp.float32)],
    )(qt, w, p_pad)


def _rev_tc(qt, w, mask):
    iblk = 1024
    jblk = 2048
    nib = P_SAMP // iblk
    njb = P_OV // jblk

    def body(qt_ref, w_ref, mask_ref, sum_out, max_out, acc, ssum, smax):
        i = pl.program_id(0)
        j = pl.program_id(1)
        d = lax.dot_general(qt_ref[...], w_ref[...], (((0,), (0,)), ((), ())),
                            precision=lax.Precision.HIGHEST,
                            preferred_element_type=jnp.float32)
        m = jnp.min(d, axis=1, keepdims=True)
        acc[...] = jnp.where(j == 0, m, jnp.minimum(acc[...], m))

        @pl.when(j == njb - 1)
        def _():
            mm = mask_ref[...] * acc[...]
            s = jnp.sum(mm)
            mx = jnp.max(mm)
            ssum[0] = jnp.where(i == 0, s, ssum[0] + s)
            smax[0] = jnp.where(i == 0, mx, jnp.maximum(smax[0], mx))
            sum_out[...] = jnp.full((1, 1), ssum[0], jnp.float32)
            max_out[...] = jnp.full((1, 1), smax[0], jnp.float32)

    return pl.pallas_call(
        body,
        grid=(nib, njb),
        in_specs=[
            pl.BlockSpec((8, iblk), lambda i, j: (0, i)),
            pl.BlockSpec((8, jblk), lambda i, j: (0, j)),
            pl.BlockSpec((iblk, 1), lambda i, j: (i, 0)),
        ],
        out_specs=[
            pl.BlockSpec((1, 1), lambda i, j: (0, 0)),
            pl.BlockSpec((1, 1), lambda i, j: (0, 0)),
        ],
        out_shape=[
            jax.ShapeDtypeStruct((1, 1), jnp.float32),
            jax.ShapeDtypeStruct((1, 1), jnp.float32),
        ],
        scratch_shapes=[
            pltpu.VMEM((iblk, 1), jnp.float32),
            pltpu.SMEM((1,), jnp.float32),
            pltpu.SMEM((1,), jnp.float32),
        ],
    )(qt, w, mask)


def _sample_consts():
    import numpy as _np
    kr = jax.random.key(42)
    kr1, kr2 = jax.random.split(kr)
    r1 = jnp.sqrt(jax.random.uniform(kr1, (N_SF, S, 1), dtype=jnp.float32))
    r2 = jax.random.uniform(kr2, (N_SF, S, 1), dtype=jnp.float32)
    pad = P_SAMP - N_SF * S
    a = _np.pad(_np.asarray((1.0 - r1).reshape(-1)), (0, pad))
    b = _np.pad(_np.asarray((r1 * (1.0 - r2)).reshape(-1)), (0, pad))
    c = _np.pad(_np.asarray((r1 * r2).reshape(-1)), (0, pad))
    m = (_np.arange(P_SAMP) < N_SF * S).astype(_np.float32).reshape(P_SAMP, 1)
    return a, b, c, m


_A_CONST, _B_CONST, _C_CONST, _MASK_CONST = _sample_consts()


def kernel(original_vertices, original_faces, simplified_vertices,
           simplified_faces, face_probabilities):
    f32 = jnp.float32
    ov_flat = jnp.pad(original_vertices.reshape(-1).astype(f32), (0, 30720 - 3 * N_OV))
    of_flat = jnp.pad(original_faces.reshape(-1).astype(jnp.int32), (0, 3 * P_OF - 3 * N_OF))
    sv_flat = jnp.pad(simplified_vertices.reshape(-1).astype(f32), (0, 2048 - 3 * N_SV))
    sf_flat = jnp.pad(simplified_faces.reshape(-1).astype(jnp.int32), (0, 3072 - 3 * N_SF))

    a_flat, b_flat, c_flat = (jnp.asarray(_A_CONST), jnp.asarray(_B_CONST),
                              jnp.asarray(_C_CONST))

    w_fwd, qt_fwd, qt_rev, w_rev = _sc_pack(
        ov_flat, of_flat, sv_flat, sf_flat, a_flat, b_flat, c_flat)

    p_pad = jnp.pad(face_probabilities.astype(f32), (0, P_Q - N_SF)).reshape(P_Q, 1)
    mask = jnp.asarray(_MASK_CONST)

    fwd_term, sum_p = _fwd_tc(qt_fwd, w_fwd, p_pad)
    rev_sum, rev_max = _rev_tc(qt_rev, w_rev, mask)

    rev_term = 0.1 * sum_p[0, 0] * rev_sum[0, 0] / rev_max[0, 0]
    return fwd_term[0, 0] + rev_term


# speedup vs baseline: 1.4651x; 1.0085x over previous
import functools

import jax
import jax.numpy as jnp
from jax import lax
from jax.experimental import pallas as pl
from jax.experimental.pallas import tpu as pltpu
from jax.experimental.pallas import tpu_sc as plsc

N_OV = 10000
N_OF = 20000
N_SV = 600
N_SF = 1000
S = 8

P_OV = 10240
P_OF = 20480
P_Q = 1024
P_SAMP = 8192

NC = 2
NS = 16
NW = NC * NS
LANES = 16

WF_PER = P_OF // NW
QF_PER = P_Q // NW
QR_PER = P_SAMP // NW
WR_PER = P_OV // NW

BIG = 1e30


def _sc_pack(ov_flat, of_flat, sv_flat, sf_flat, a_flat, b_flat, c_flat):
    mesh = plsc.VectorSubcoreMesh(core_axis_name="c", subcore_axis_name="s")

    @functools.partial(
        pl.kernel,
        mesh=mesh,
        compiler_params=pltpu.CompilerParams(needs_layout_passes=False),
        out_type=[
            jax.ShapeDtypeStruct((8, P_OF), jnp.float32),
            jax.ShapeDtypeStruct((8, P_Q), jnp.float32),
            jax.ShapeDtypeStruct((8, P_SAMP), jnp.float32),
            jax.ShapeDtypeStruct((8, P_OV), jnp.float32),
        ],
        scratch_types=[
            pltpu.VMEM((30720,), jnp.float32),
            pltpu.VMEM((3 * WF_PER,), jnp.int32),
            pltpu.VMEM((2048,), jnp.float32),
            pltpu.VMEM((3072,), jnp.int32),
            pltpu.VMEM((QR_PER,), jnp.float32),
            pltpu.VMEM((QR_PER,), jnp.float32),
            pltpu.VMEM((QR_PER,), jnp.float32),
            pltpu.VMEM((8, WF_PER), jnp.float32),
            pltpu.VMEM((8, 128), jnp.float32),
            pltpu.VMEM((8, QR_PER), jnp.float32),
            pltpu.VMEM((8, 128), jnp.float32),
        ],
    )
    def body(ov_h, of_h, sv_h, sf_h, a_h, b_h, c_h,
             wf_h, qf_h, qr_h, wr_h,
             ov_v, of_v, sv_v, sf_v, a_v, b_v, c_v,
             wf_s, qf_s, qr_s, wr_s):
        wid = lax.axis_index("s") * NC + lax.axis_index("c")

        pltpu.sync_copy(ov_h, ov_v)
        pltpu.sync_copy(of_h.at[pl.ds(wid * 3 * WF_PER, 3 * WF_PER)], of_v)
        pltpu.sync_copy(sv_h, sv_v)
        pltpu.sync_copy(sf_h, sf_v)
        pltpu.sync_copy(a_h.at[pl.ds(wid * QR_PER, QR_PER)], a_v)
        pltpu.sync_copy(b_h.at[pl.ds(wid * QR_PER, QR_PER)], b_v)
        pltpu.sync_copy(c_h.at[pl.ds(wid * QR_PER, QR_PER)], c_v)

        iota = lax.iota(jnp.int32, LANES)
        ones = jnp.full((LANES,), 1.0, jnp.float32)
        zeros = jnp.zeros((LANES,), jnp.float32)

        def gat_f(ref, idx):
            return plsc.load_gather(ref, [idx])

        def tri_coords(faces_ref, fidx):
            g0 = plsc.load_gather(faces_ref, [fidx * 3])
            g1 = plsc.load_gather(faces_ref, [fidx * 3 + 1])
            g2 = plsc.load_gather(faces_ref, [fidx * 3 + 2])
            return g0, g1, g2

        def wf_body(i, carry):
            base = i * LANES
            col_l = base + iota
            g0, g1, g2 = tri_coords(of_v, col_l)
            third = jnp.full((LANES,), 1.0 / 3.0, jnp.float32)
            bx = (gat_f(ov_v, g0 * 3) + gat_f(ov_v, g1 * 3) + gat_f(ov_v, g2 * 3)) * third
            by = (gat_f(ov_v, g0 * 3 + 1) + gat_f(ov_v, g1 * 3 + 1) + gat_f(ov_v, g2 * 3 + 1)) * third
            bz = (gat_f(ov_v, g0 * 3 + 2) + gat_f(ov_v, g1 * 3 + 2) + gat_f(ov_v, g2 * 3 + 2)) * third
            col_g = col_l + wid * WF_PER
            nrm = bx * bx + by * by + bz * bz
            nrm = jnp.where(col_g < N_OF, nrm, BIG)
            sl = pl.ds(base, LANES)
            wf_s[0, sl] = -2.0 * bx
            wf_s[1, sl] = -2.0 * by
            wf_s[2, sl] = -2.0 * bz
            wf_s[3, sl] = ones
            wf_s[4, sl] = nrm
            wf_s[5, sl] = zeros
            wf_s[6, sl] = zeros
            wf_s[7, sl] = zeros
            return carry

        lax.fori_loop(0, WF_PER // LANES, wf_body, 0)

        @pl.when(wid < 8)
        def _qt_fwd():
            def qf_body(i, carry):
                base = i * LANES
                col_g = base + iota + wid * 128
                g0, g1, g2 = tri_coords(sf_v, col_g)
                third = jnp.full((LANES,), 1.0 / 3.0, jnp.float32)
                bx = (gat_f(sv_v, g0 * 3) + gat_f(sv_v, g1 * 3) + gat_f(sv_v, g2 * 3)) * third
                by = (gat_f(sv_v, g0 * 3 + 1) + gat_f(sv_v, g1 * 3 + 1) + gat_f(sv_v, g2 * 3 + 1)) * third
                bz = (gat_f(sv_v, g0 * 3 + 2) + gat_f(sv_v, g1 * 3 + 2) + gat_f(sv_v, g2 * 3 + 2)) * third
                sl = pl.ds(base, LANES)
                qf_s[0, sl] = bx
                qf_s[1, sl] = by
                qf_s[2, sl] = bz
                qf_s[3, sl] = bx * bx + by * by + bz * bz
                qf_s[4, sl] = ones
                qf_s[5, sl] = zeros
                qf_s[6, sl] = zeros
                qf_s[7, sl] = zeros
                return carry

            lax.fori_loop(0, 128 // LANES, qf_body, 0)
            pltpu.sync_copy(qf_s, qf_h.at[:, pl.ds(wid * 128, 128)])

        def qr_body(i, carry):
            base = i * LANES
            col_l = base + iota
            col_g = col_l + wid * QR_PER
            f = lax.shift_right_logical(col_g, 3)
            g0, g1, g2 = tri_coords(sf_v, f)
            sl = pl.ds(base, LANES)
            av = a_v[sl]
            bv = b_v[sl]
            cv = c_v[sl]
            sx = av * gat_f(sv_v, g0 * 3) + bv * gat_f(sv_v, g1 * 3) + cv * gat_f(sv_v, g2 * 3)
            sy = av * gat_f(sv_v, g0 * 3 + 1) + bv * gat_f(sv_v, g1 * 3 + 1) + cv * gat_f(sv_v, g2 * 3 + 1)
            sz = av * gat_f(sv_v, g0 * 3 + 2) + bv * gat_f(sv_v, g1 * 3 + 2) + cv * gat_f(sv_v, g2 * 3 + 2)
            qr_s[0, sl] = sx
            qr_s[1, sl] = sy
            qr_s[2, sl] = sz
            qr_s[3, sl] = sx * sx + sy * sy + sz * sz
            qr_s[4, sl] = ones
            qr_s[5, sl] = zeros
            qr_s[6, sl] = zeros
            qr_s[7, sl] = zeros
            return carry

        lax.fori_loop(0, QR_PER // LANES, qr_body, 0)

        for k in range(3):
            chunk = wid + 32 * k

            @pl.when(chunk < 80)
            def _wr_chunk(chunk=chunk):
                def wr_body(i, carry):
                    base = i * LANES
                    col_g = base + iota + chunk * 128
                    vx = gat_f(ov_v, col_g * 3)
                    vy = gat_f(ov_v, col_g * 3 + 1)
                    vz = gat_f(ov_v, col_g * 3 + 2)
                    nrm = vx * vx + vy * vy + vz * vz
                    nrm = jnp.where(col_g < N_OV, nrm, BIG)
                    sl = pl.ds(base, LANES)
                    wr_s[0, sl] = -2.0 * vx
                    wr_s[1, sl] = -2.0 * vy
                    wr_s[2, sl] = -2.0 * vz
                    wr_s[3, sl] = ones
                    wr_s[4, sl] = nrm
                    wr_s[5, sl] = zeros
                    wr_s[6, sl] = zeros
                    wr_s[7, sl] = zeros
                    return carry

                lax.fori_loop(0, 128 // LANES, wr_body, 0)
                pltpu.sync_copy(wr_s, wr_h.at[:, pl.ds(chunk * 128, 128)])

        pltpu.sync_copy(wf_s, wf_h.at[:, pl.ds(wid * WF_PER, WF_PER)])
        pltpu.sync_copy(qr_s, qr_h.at[:, pl.ds(wid * QR_PER, QR_PER)])

    return body(ov_flat, of_flat, sv_flat, sf_flat, a_flat, b_flat, c_flat)


def _fwd_tc(qt, w, p_pad):
    jblk = 2048
    njb = P_OF // jblk

    def body(qt_ref, w_ref, p_ref, fwd_out, sump_out, acc):
        j = pl.program_id(0)
        d = lax.dot_general(qt_ref[...], w_ref[...], (((0,), (0,)), ((), ())),
                            precision=lax.Precision.HIGHEST,
                            preferred_element_type=jnp.float32)
        m = d[:, 0:128]
        for k in range(1, jblk // 128):
            m = jnp.minimum(m, d[:, k * 128:(k + 1) * 128])
        acc[...] = jnp.where(j == 0, m, jnp.minimum(acc[...], m))

        @pl.when(j == njb - 1)
        def _():
            p = p_ref[...]
            sp = jnp.sum(p)
            row_min = jnp.min(acc[...], axis=1, keepdims=True)
            fwd_out[...] = jnp.full((1, 1), jnp.sum(p * row_min) + 1e-4 * (float(N_SF) - sp), jnp.float32)
            sump_out[...] = jnp.full((1, 1), sp, jnp.float32)

    return pl.pallas_call(
        body,
        grid=(njb,),
        in_specs=[
            pl.BlockSpec((8, P_Q), lambda j: (0, 0)),
            pl.BlockSpec((8, jblk), lambda j: (0, j)),
            pl.BlockSpec((P_Q, 1), lambda j: (0, 0)),
        ],
        out_specs=[
            pl.BlockSpec((1, 1), lambda j: (0, 0)),
            pl.BlockSpec((1, 1), lambda j: (0, 0)),
        ],
        out_shape=[
            jax.ShapeDtypeStruct((1, 1), jnp.float32),
            jax.ShapeDtypeStruct((1, 1), jnp.float32),
        ],
        scratch_shapes=[pltpu.VMEM((P_Q, 128), jnp.float32)],
    )(qt, w, p_pad)


def _rev_tc(qt, w, mask):
    iblk = 1024
    jblk = 2048
    nib = P_SAMP // iblk
    njb = P_OV // jblk

    def body(qt_ref, w_ref, mask_ref, sum_out, max_out, acc, ssum, smax):
        i = pl.program_id(0)
        j = pl.program_id(1)
        d = lax.dot_general(qt_ref[...], w_ref[...], (((0,), (0,)), ((), ())),
                            precision=lax.Precision.HIGHEST,
                            preferred_element_type=jnp.float32)
        m = d[:, 0:128]
        for k in range(1, jblk // 128):
            m = jnp.minimum(m, d[:, k * 128:(k + 1) * 128])
        acc[...] = jnp.where(j == 0, m, jnp.minimum(acc[...], m))

        @pl.when(j == njb - 1)
        def _():
            mm = mask_ref[...] * jnp.min(acc[...], axis=1, keepdims=True)
            s = jnp.sum(mm)
            mx = jnp.max(mm)
            ssum[0] = jnp.where(i == 0, s, ssum[0] + s)
            smax[0] = jnp.where(i == 0, mx, jnp.maximum(smax[0], mx))
            sum_out[...] = jnp.full((1, 1), ssum[0], jnp.float32)
            max_out[...] = jnp.full((1, 1), smax[0], jnp.float32)

    return pl.pallas_call(
        body,
        grid=(nib, njb),
        in_specs=[
            pl.BlockSpec((8, iblk), lambda i, j: (0, i)),
            pl.BlockSpec((8, jblk), lambda i, j: (0, j)),
            pl.BlockSpec((iblk, 1), lambda i, j: (i, 0)),
        ],
        out_specs=[
            pl.BlockSpec((1, 1), lambda i, j: (0, 0)),
            pl.BlockSpec((1, 1), lambda i, j: (0, 0)),
        ],
        out_shape=[
            jax.ShapeDtypeStruct((1, 1), jnp.float32),
            jax.ShapeDtypeStruct((1, 1), jnp.float32),
        ],
        scratch_shapes=[
            pltpu.VMEM((iblk, 128), jnp.float32),
            pltpu.SMEM((1,), jnp.float32),
            pltpu.SMEM((1,), jnp.float32),
        ],
    )(qt, w, mask)


def _sample_consts():
    import numpy as _np
    kr = jax.random.key(42)
    kr1, kr2 = jax.random.split(kr)
    r1 = jnp.sqrt(jax.random.uniform(kr1, (N_SF, S, 1), dtype=jnp.float32))
    r2 = jax.random.uniform(kr2, (N_SF, S, 1), dtype=jnp.float32)
    pad = P_SAMP - N_SF * S
    a = _np.pad(_np.asarray((1.0 - r1).reshape(-1)), (0, pad))
    b = _np.pad(_np.asarray((r1 * (1.0 - r2)).reshape(-1)), (0, pad))
    c = _np.pad(_np.asarray((r1 * r2).reshape(-1)), (0, pad))
    m = (_np.arange(P_SAMP) < N_SF * S).astype(_np.float32).reshape(P_SAMP, 1)
    return a, b, c, m


_A_CONST, _B_CONST, _C_CONST, _MASK_CONST = _sample_consts()


def kernel(original_vertices, original_faces, simplified_vertices,
           simplified_faces, face_probabilities):
    f32 = jnp.float32
    ov_flat = jnp.pad(original_vertices.reshape(-1).astype(f32), (0, 30720 - 3 * N_OV))
    of_flat = jnp.pad(original_faces.reshape(-1).astype(jnp.int32), (0, 3 * P_OF - 3 * N_OF))
    sv_flat = jnp.pad(simplified_vertices.reshape(-1).astype(f32), (0, 2048 - 3 * N_SV))
    sf_flat = jnp.pad(simplified_faces.reshape(-1).astype(jnp.int32), (0, 3072 - 3 * N_SF))

    a_flat, b_flat, c_flat = (jnp.asarray(_A_CONST), jnp.asarray(_B_CONST),
                              jnp.asarray(_C_CONST))

    w_fwd, qt_fwd, qt_rev, w_rev = _sc_pack(
        ov_flat, of_flat, sv_flat, sf_flat, a_flat, b_flat, c_flat)

    p_pad = jnp.pad(face_probabilities.astype(f32), (0, P_Q - N_SF)).reshape(P_Q, 1)
    mask = jnp.asarray(_MASK_CONST)

    fwd_term, sum_p = _fwd_tc(qt_fwd, w_fwd, p_pad)
    rev_sum, rev_max = _rev_tc(qt_rev, w_rev, mask)

    rev_term = 0.1 * sum_p[0, 0] * rev_sum[0, 0] / rev_max[0, 0]
    return fwd_term[0, 0] + rev_term


# speedup vs baseline: 2.3891x; 1.6307x over previous
import functools

import numpy as _np

import jax
import jax.numpy as jnp
from jax import lax
from jax.experimental import pallas as pl
from jax.experimental.pallas import tpu as pltpu
from jax.experimental.pallas import tpu_sc as plsc

N_OV = 10000
N_OF = 20000
N_SV = 600
N_SF = 1000
S = 8

P_OV = 10240
P_OF = 20480
P_Q = 1024
P_SAMP = 8192

NC = 2
NS = 16
NW = NC * NS
LANES = 16

WF_PER = P_OF // NW
QF_PER = P_Q // NW
QR_PER = P_SAMP // NW
WR_PER = P_OV // NW

BIG = 1e30


def _sc_pack(ov_flat, of_flat, sv_flat, sf_flat, a_flat, b_flat, c_flat):
    mesh = plsc.VectorSubcoreMesh(core_axis_name="c", subcore_axis_name="s")

    @functools.partial(
        pl.kernel,
        mesh=mesh,
        compiler_params=pltpu.CompilerParams(needs_layout_passes=False),
        out_type=[
            jax.ShapeDtypeStruct((8, P_OF), jnp.float32),
            jax.ShapeDtypeStruct((8, P_Q), jnp.float32),
            jax.ShapeDtypeStruct((8, P_SAMP), jnp.float32),
            jax.ShapeDtypeStruct((8, P_OV), jnp.float32),
            jax.ShapeDtypeStruct((P_OF,), jnp.float32),
            jax.ShapeDtypeStruct((P_Q,), jnp.float32),
            jax.ShapeDtypeStruct((P_SAMP,), jnp.float32),
            jax.ShapeDtypeStruct((P_OV,), jnp.float32),
        ],
        scratch_types=[
            pltpu.VMEM((30720,), jnp.float32),
            pltpu.VMEM((3 * WF_PER,), jnp.int32),
            pltpu.VMEM((2048,), jnp.float32),
            pltpu.VMEM((3072,), jnp.int32),
            pltpu.VMEM((QR_PER,), jnp.float32),
            pltpu.VMEM((QR_PER,), jnp.float32),
            pltpu.VMEM((QR_PER,), jnp.float32),
            pltpu.VMEM((8, WF_PER), jnp.float32),
            pltpu.VMEM((8, 128), jnp.float32),
            pltpu.VMEM((8, QR_PER), jnp.float32),
            pltpu.VMEM((8, 128), jnp.float32),
            pltpu.VMEM((WF_PER,), jnp.float32),
            pltpu.VMEM((128,), jnp.float32),
            pltpu.VMEM((QR_PER,), jnp.float32),
            pltpu.VMEM((128,), jnp.float32),
        ],
    )
    def body(ov_h, of_h, sv_h, sf_h, a_h, b_h, c_h,
             wf_h, qf_h, qr_h, wr_h, vnf_h, qnf_h, qnr_h, vnr_h,
             ov_v, of_v, sv_v, sf_v, a_v, b_v, c_v,
             wf_s, qf_s, qr_s, wr_s, vnf_s, qnf_s, qnr_s, vnr_s):
        wid = lax.axis_index("s") * NC + lax.axis_index("c")

        pltpu.sync_copy(ov_h, ov_v)
        pltpu.sync_copy(of_h.at[pl.ds(wid * 3 * WF_PER, 3 * WF_PER)], of_v)
        pltpu.sync_copy(sv_h, sv_v)
        pltpu.sync_copy(sf_h, sf_v)
        pltpu.sync_copy(a_h.at[pl.ds(wid * QR_PER, QR_PER)], a_v)
        pltpu.sync_copy(b_h.at[pl.ds(wid * QR_PER, QR_PER)], b_v)
        pltpu.sync_copy(c_h.at[pl.ds(wid * QR_PER, QR_PER)], c_v)

        iota = lax.iota(jnp.int32, LANES)
        ones = jnp.full((LANES,), 1.0, jnp.float32)
        zeros = jnp.zeros((LANES,), jnp.float32)

        def gat_f(ref, idx):
            return plsc.load_gather(ref, [idx])

        def tri_coords(faces_ref, fidx):
            g0 = plsc.load_gather(faces_ref, [fidx * 3])
            g1 = plsc.load_gather(faces_ref, [fidx * 3 + 1])
            g2 = plsc.load_gather(faces_ref, [fidx * 3 + 2])
            return g0, g1, g2

        def wf_body(i, carry):
            base = i * LANES
            col_l = base + iota
            g0, g1, g2 = tri_coords(of_v, col_l)
            third = jnp.full((LANES,), 1.0 / 3.0, jnp.float32)
            bx = (gat_f(ov_v, g0 * 3) + gat_f(ov_v, g1 * 3) + gat_f(ov_v, g2 * 3)) * third
            by = (gat_f(ov_v, g0 * 3 + 1) + gat_f(ov_v, g1 * 3 + 1) + gat_f(ov_v, g2 * 3 + 1)) * third
            bz = (gat_f(ov_v, g0 * 3 + 2) + gat_f(ov_v, g1 * 3 + 2) + gat_f(ov_v, g2 * 3 + 2)) * third
            col_g = col_l + wid * WF_PER
            nrm = bx * bx + by * by + bz * bz
            nrm = jnp.where(col_g < N_OF, nrm, BIG)
            sl = pl.ds(base, LANES)
            wf_s[0, sl] = -2.0 * bx
            wf_s[1, sl] = -2.0 * by
            wf_s[2, sl] = -2.0 * bz
            wf_s[3, sl] = zeros
            wf_s[4, sl] = zeros
            wf_s[5, sl] = zeros
            wf_s[6, sl] = zeros
            wf_s[7, sl] = zeros
            vnf_s[sl] = nrm
            return carry

        lax.fori_loop(0, WF_PER // LANES, wf_body, 0)

        @pl.when(wid < 8)
        def _qt_fwd():
            def qf_body(i, carry):
                base = i * LANES
                col_g = base + iota + wid * 128
                g0, g1, g2 = tri_coords(sf_v, col_g)
                third = jnp.full((LANES,), 1.0 / 3.0, jnp.float32)
                bx = (gat_f(sv_v, g0 * 3) + gat_f(sv_v, g1 * 3) + gat_f(sv_v, g2 * 3)) * third
                by = (gat_f(sv_v, g0 * 3 + 1) + gat_f(sv_v, g1 * 3 + 1) + gat_f(sv_v, g2 * 3 + 1)) * third
                bz = (gat_f(sv_v, g0 * 3 + 2) + gat_f(sv_v, g1 * 3 + 2) + gat_f(sv_v, g2 * 3 + 2)) * third
                sl = pl.ds(base, LANES)
                qf_s[0, sl] = bx
                qf_s[1, sl] = by
                qf_s[2, sl] = bz
                qf_s[3, sl] = zeros
                qf_s[4, sl] = zeros
                qf_s[5, sl] = zeros
                qf_s[6, sl] = zeros
                qf_s[7, sl] = zeros
                qnf_s[sl] = bx * bx + by * by + bz * bz
                return carry

            lax.fori_loop(0, 128 // LANES, qf_body, 0)
            pltpu.sync_copy(qf_s, qf_h.at[:, pl.ds(wid * 128, 128)])
            pltpu.sync_copy(qnf_s, qnf_h.at[pl.ds(wid * 128, 128)])

        def qr_body(i, carry):
            base = i * LANES
            col_l = base + iota
            col_g = col_l + wid * QR_PER
            f = lax.shift_right_logical(col_g, 3)
            g0, g1, g2 = tri_coords(sf_v, f)
            sl = pl.ds(base, LANES)
            av = a_v[sl]
            bv = b_v[sl]
            cv = c_v[sl]
            sx = av * gat_f(sv_v, g0 * 3) + bv * gat_f(sv_v, g1 * 3) + cv * gat_f(sv_v, g2 * 3)
            sy = av * gat_f(sv_v, g0 * 3 + 1) + bv * gat_f(sv_v, g1 * 3 + 1) + cv * gat_f(sv_v, g2 * 3 + 1)
            sz = av * gat_f(sv_v, g0 * 3 + 2) + bv * gat_f(sv_v, g1 * 3 + 2) + cv * gat_f(sv_v, g2 * 3 + 2)
            qr_s[0, sl] = sx
            qr_s[1, sl] = sy
            qr_s[2, sl] = sz
            qr_s[3, sl] = zeros
            qr_s[4, sl] = zeros
            qr_s[5, sl] = zeros
            qr_s[6, sl] = zeros
            qr_s[7, sl] = zeros
            qnr_s[sl] = sx * sx + sy * sy + sz * sz
            return carry

        lax.fori_loop(0, QR_PER // LANES, qr_body, 0)

        for k in range(3):
            chunk = wid + 32 * k

            @pl.when(chunk < 80)
            def _wr_chunk(chunk=chunk):
                def wr_body(i, carry):
                    base = i * LANES
                    col_g = base + iota + chunk * 128
                    vx = gat_f(ov_v, col_g * 3)
                    vy = gat_f(ov_v, col_g * 3 + 1)
                    vz = gat_f(ov_v, col_g * 3 + 2)
                    nrm = vx * vx + vy * vy + vz * vz
                    nrm = jnp.where(col_g < N_OV, nrm, BIG)
                    sl = pl.ds(base, LANES)
                    wr_s[0, sl] = -2.0 * vx
                    wr_s[1, sl] = -2.0 * vy
                    wr_s[2, sl] = -2.0 * vz
                    wr_s[3, sl] = zeros
                    wr_s[4, sl] = zeros
                    wr_s[5, sl] = zeros
                    wr_s[6, sl] = zeros
                    wr_s[7, sl] = zeros
                    vnr_s[sl] = nrm
                    return carry

                lax.fori_loop(0, 128 // LANES, wr_body, 0)
                pltpu.sync_copy(wr_s, wr_h.at[:, pl.ds(chunk * 128, 128)])
                pltpu.sync_copy(vnr_s, vnr_h.at[pl.ds(chunk * 128, 128)])

        pltpu.sync_copy(wf_s, wf_h.at[:, pl.ds(wid * WF_PER, WF_PER)])
        pltpu.sync_copy(qr_s, qr_h.at[:, pl.ds(wid * QR_PER, QR_PER)])
        pltpu.sync_copy(vnf_s, vnf_h.at[pl.ds(wid * WF_PER, WF_PER)])
        pltpu.sync_copy(qnr_s, qnr_h.at[pl.ds(wid * QR_PER, QR_PER)])

    return body(ov_flat, of_flat, sv_flat, sf_flat, a_flat, b_flat, c_flat)


def _fwd_tc(qt, w, vn, qn, p_pad):
    jblk = 2048
    njb = P_OF // jblk
    dn = (((0,), (0,)), ((), ()))

    def body(qt_ref, w_ref, vn_ref, qn_ref, p_ref, fwd_out, sump_out, acc):
        j = pl.program_id(0)
        qtv = qt_ref[...]
        qh = qtv.astype(jnp.bfloat16)
        ql = (qtv - qh.astype(jnp.float32)).astype(jnp.bfloat16)
        wv = w_ref[...]
        wh = wv.astype(jnp.bfloat16)
        wl = (wv - wh.astype(jnp.float32)).astype(jnp.bfloat16)
        da = lax.dot_general(qh, wh, dn, preferred_element_type=jnp.float32)
        db = lax.dot_general(qh, wl, dn, preferred_element_type=jnp.float32)
        dc = lax.dot_general(ql, wh, dn, preferred_element_type=jnp.float32)
        vnr = vn_ref[...]
        m = None
        for k in range(jblk // 128):
            sl = slice(k * 128, (k + 1) * 128)
            chunk = (da[:, sl] + db[:, sl] + dc[:, sl]) + vnr[:, sl]
            m = chunk if m is None else jnp.minimum(m, chunk)
        acc[...] = jnp.where(j == 0, m, jnp.minimum(acc[...], m))

        @pl.when(j == njb - 1)
        def _():
            p = p_ref[...]
            sp = jnp.sum(p)
            row_min = jnp.min(acc[...], axis=1, keepdims=True) + qn_ref[...]
            fwd_out[...] = jnp.full((1, 1), jnp.sum(p * row_min) + 1e-4 * (float(N_SF) - sp), jnp.float32)
            sump_out[...] = jnp.full((1, 1), sp, jnp.float32)

    return pl.pallas_call(
        body,
        grid=(njb,),
        in_specs=[
            pl.BlockSpec((8, P_Q), lambda j: (0, 0)),
            pl.BlockSpec((8, jblk), lambda j: (0, j)),
            pl.BlockSpec((1, jblk), lambda j: (0, j)),
            pl.BlockSpec((P_Q, 1), lambda j: (0, 0)),
            pl.BlockSpec((P_Q, 1), lambda j: (0, 0)),
        ],
        out_specs=[
            pl.BlockSpec((1, 1), lambda j: (0, 0)),
            pl.BlockSpec((1, 1), lambda j: (0, 0)),
        ],
        out_shape=[
            jax.ShapeDtypeStruct((1, 1), jnp.float32),
            jax.ShapeDtypeStruct((1, 1), jnp.float32),
        ],
        scratch_shapes=[pltpu.VMEM((P_Q, 128), jnp.float32)],
    )(qt, w, vn, qn, p_pad)


def _rev_tc(qt, w, vn, qn, mask):
    iblk = 1024
    jblk = 2048
    nib = P_SAMP // iblk
    njb = P_OV // jblk
    dn = (((0,), (0,)), ((), ()))

    def body(qt_ref, w_ref, vn_ref, qn_ref, mask_ref, sum_out, max_out, acc, ssum, smax):
        i = pl.program_id(0)
        j = pl.program_id(1)
        qtv = qt_ref[...]
        qh = qtv.astype(jnp.bfloat16)
        ql = (qtv - qh.astype(jnp.float32)).astype(jnp.bfloat16)
        wv = w_ref[...]
        wh = wv.astype(jnp.bfloat16)
        wl = (wv - wh.astype(jnp.float32)).astype(jnp.bfloat16)
        da = lax.dot_general(qh, wh, dn, preferred_element_type=jnp.float32)
        db = lax.dot_general(qh, wl, dn, preferred_element_type=jnp.float32)
        dc = lax.dot_general(ql, wh, dn, preferred_element_type=jnp.float32)
        vnr = vn_ref[...]
        m = None
        for k in range(jblk // 128):
            sl = slice(k * 128, (k + 1) * 128)
            chunk = (da[:, sl] + db[:, sl] + dc[:, sl]) + vnr[:, sl]
            m = chunk if m is None else jnp.minimum(m, chunk)
        acc[...] = jnp.where(j == 0, m, jnp.minimum(acc[...], m))

        @pl.when(j == njb - 1)
        def _():
            row_min = jnp.min(acc[...], axis=1, keepdims=True) + qn_ref[...]
            mm = mask_ref[...] * row_min
            s = jnp.sum(mm)
            mx = jnp.max(mm)
            ssum[0] = jnp.where(i == 0, s, ssum[0] + s)
            smax[0] = jnp.where(i == 0, mx, jnp.maximum(smax[0], mx))
            sum_out[...] = jnp.full((1, 1), ssum[0], jnp.float32)
            max_out[...] = jnp.full((1, 1), smax[0], jnp.float32)

    return pl.pallas_call(
        body,
        grid=(nib, njb),
        in_specs=[
            pl.BlockSpec((8, iblk), lambda i, j: (0, i)),
            pl.BlockSpec((8, jblk), lambda i, j: (0, j)),
            pl.BlockSpec((1, jblk), lambda i, j: (0, j)),
            pl.BlockSpec((iblk, 1), lambda i, j: (i, 0)),
            pl.BlockSpec((iblk, 1), lambda i, j: (i, 0)),
        ],
        out_specs=[
            pl.BlockSpec((1, 1), lambda i, j: (0, 0)),
            pl.BlockSpec((1, 1), lambda i, j: (0, 0)),
        ],
        out_shape=[
            jax.ShapeDtypeStruct((1, 1), jnp.float32),
            jax.ShapeDtypeStruct((1, 1), jnp.float32),
        ],
        scratch_shapes=[
            pltpu.VMEM((iblk, 128), jnp.float32),
            pltpu.SMEM((1,), jnp.float32),
            pltpu.SMEM((1,), jnp.float32),
        ],
    )(qt, w, vn, qn, mask)


def _tf2x32(k1, k2, x0, x1):
    def rotl(x, d):
        return ((x << _np.uint32(d)) | (x >> _np.uint32(32 - d))).astype(_np.uint32)
    rot = [(13, 15, 26, 6), (17, 29, 16, 24)]
    ks = [_np.uint32(k1), _np.uint32(k2),
          _np.uint32(k1 ^ k2 ^ _np.uint32(0x1BD11BDA))]
    x = [(x0 + ks[0]).astype(_np.uint32), (x1 + ks[1]).astype(_np.uint32)]
    for i, (rs, ka, kb) in enumerate([(rot[0], 1, 2), (rot[1], 2, 0),
                                      (rot[0], 0, 1), (rot[1], 1, 2), (rot[0], 2, 0)]):
        for r in rs:
            x[0] = (x[0] + x[1]).astype(_np.uint32)
            x[1] = rotl(x[1], r)
            x[1] = (x[1] ^ x[0]).astype(_np.uint32)
        x[0] = (x[0] + ks[ka]).astype(_np.uint32)
        x[1] = (x[1] + ks[kb] + _np.uint32(i + 1)).astype(_np.uint32)
    return x[0], x[1]


def _tf_uniform(key, n):
    b1, b2 = _tf2x32(key[0], key[1], _np.zeros(n, _np.uint32),
                     _np.arange(n, dtype=_np.uint32))
    bits = b1 ^ b2
    flt = ((bits >> _np.uint32(9)) | _np.uint32(0x3F800000)).view(_np.float32)
    return _np.maximum(_np.float32(0.0), flt - _np.float32(1.0))


def _sample_consts():
    key = _np.array([0, 42], _np.uint32)
    b1, b2 = _tf2x32(key[0], key[1], _np.zeros(2, _np.uint32),
                     _np.arange(2, dtype=_np.uint32))
    k1, k2 = _np.stack([b1, b2], axis=1)
    r1 = _np.sqrt(_tf_uniform(k1, N_SF * S))
    r2 = _tf_uniform(k2, N_SF * S)
    pad = P_SAMP - N_SF * S
    a = _np.pad((_np.float32(1.0) - r1), (0, pad))
    b = _np.pad((r1 * (_np.float32(1.0) - r2)), (0, pad))
    c = _np.pad((r1 * r2), (0, pad))
    m = (_np.arange(P_SAMP) < N_SF * S).astype(_np.float32).reshape(P_SAMP, 1)
    return a, b, c, m


_A_CONST, _B_CONST, _C_CONST, _MASK_CONST = _sample_consts()


def kernel(original_vertices, original_faces, simplified_vertices,
           simplified_faces, face_probabilities):
    f32 = jnp.float32
    ov_flat = jnp.pad(original_vertices.reshape(-1).astype(f32), (0, 30720 - 3 * N_OV))
    of_flat = jnp.pad(original_faces.reshape(-1).astype(jnp.int32), (0, 3 * P_OF - 3 * N_OF))
    sv_flat = jnp.pad(simplified_vertices.reshape(-1).astype(f32), (0, 2048 - 3 * N_SV))
    sf_flat = jnp.pad(simplified_faces.reshape(-1).astype(jnp.int32), (0, 3072 - 3 * N_SF))

    a_flat, b_flat, c_flat = (jnp.asarray(_A_CONST), jnp.asarray(_B_CONST),
                              jnp.asarray(_C_CONST))

    (w_fwd, qt_fwd, qt_rev, w_rev,
     vn_fwd, qn_fwd, qn_rev, vn_rev) = _sc_pack(
        ov_flat, of_flat, sv_flat, sf_flat, a_flat, b_flat, c_flat)

    p_pad = jnp.pad(face_probabilities.astype(f32), (0, P_Q - N_SF)).reshape(P_Q, 1)
    mask = jnp.asarray(_MASK_CONST)

    fwd_term, sum_p = _fwd_tc(qt_fwd, w_fwd, vn_fwd.reshape(1, P_OF),
                              qn_fwd.reshape(P_Q, 1), p_pad)
    rev_sum, rev_max = _rev_tc(qt_rev, w_rev, vn_rev.reshape(1, P_OV),
                               qn_rev.reshape(P_SAMP, 1), mask)

    rev_term = 0.1 * sum_p[0, 0] * rev_sum[0, 0] / rev_max[0, 0]
    return fwd_term[0, 0] + rev_term


# speedup vs baseline: 2.9375x; 1.2295x over previous
import functools

import numpy as _np

import jax
import jax.numpy as jnp
from jax import lax
from jax.experimental import pallas as pl
from jax.experimental.pallas import tpu as pltpu
from jax.experimental.pallas import tpu_sc as plsc

N_OV = 10000
N_OF = 20000
N_SV = 600
N_SF = 1000
S = 8

P_OV = 10240
P_OF = 20480
P_Q = 1024
P_SAMP = 8192

NC = 2
NS = 16
NW = NC * NS
LANES = 16

WF_PER = P_OF // NW
QF_PER = P_Q // NW
QR_PER = P_SAMP // NW
WR_PER = P_OV // NW

BIG = 1e30


def _sc_pack(ov_flat, of_flat, sv_flat, sf_flat, a_flat, b_flat, c_flat):
    mesh = plsc.VectorSubcoreMesh(core_axis_name="c", subcore_axis_name="s")

    @functools.partial(
        pl.kernel,
        mesh=mesh,
        compiler_params=pltpu.CompilerParams(needs_layout_passes=False),
        out_type=[
            jax.ShapeDtypeStruct((8, P_OF), jnp.float32),
            jax.ShapeDtypeStruct((8, P_Q), jnp.float32),
            jax.ShapeDtypeStruct((8, P_SAMP), jnp.float32),
            jax.ShapeDtypeStruct((8, P_OV), jnp.float32),
            jax.ShapeDtypeStruct((P_OF,), jnp.float32),
            jax.ShapeDtypeStruct((P_Q,), jnp.float32),
            jax.ShapeDtypeStruct((P_SAMP,), jnp.float32),
            jax.ShapeDtypeStruct((P_OV,), jnp.float32),
        ],
        scratch_types=[
            pltpu.VMEM((30720,), jnp.float32),
            pltpu.VMEM((3 * WF_PER,), jnp.int32),
            pltpu.VMEM((2048,), jnp.float32),
            pltpu.VMEM((3072,), jnp.int32),
            pltpu.VMEM((QR_PER,), jnp.float32),
            pltpu.VMEM((QR_PER,), jnp.float32),
            pltpu.VMEM((QR_PER,), jnp.float32),
            pltpu.VMEM((8, WF_PER), jnp.float32),
            pltpu.VMEM((8, 128), jnp.float32),
            pltpu.VMEM((8, QR_PER), jnp.float32),
            pltpu.VMEM((8, 128), jnp.float32),
            pltpu.VMEM((WF_PER,), jnp.float32),
            pltpu.VMEM((128,), jnp.float32),
            pltpu.VMEM((QR_PER,), jnp.float32),
            pltpu.VMEM((128,), jnp.float32),
        ],
    )
    def body(ov_h, of_h, sv_h, sf_h, a_h, b_h, c_h,
             wf_h, qf_h, qr_h, wr_h, vnf_h, qnf_h, qnr_h, vnr_h,
             ov_v, of_v, sv_v, sf_v, a_v, b_v, c_v,
             wf_s, qf_s, qr_s, wr_s, vnf_s, qnf_s, qnr_s, vnr_s):
        wid = lax.axis_index("s") * NC + lax.axis_index("c")

        pltpu.sync_copy(ov_h, ov_v)
        pltpu.sync_copy(of_h.at[pl.ds(wid * 3 * WF_PER, 3 * WF_PER)], of_v)
        pltpu.sync_copy(sv_h, sv_v)
        pltpu.sync_copy(sf_h, sf_v)
        pltpu.sync_copy(a_h.at[pl.ds(wid * QR_PER, QR_PER)], a_v)
        pltpu.sync_copy(b_h.at[pl.ds(wid * QR_PER, QR_PER)], b_v)
        pltpu.sync_copy(c_h.at[pl.ds(wid * QR_PER, QR_PER)], c_v)

        iota = lax.iota(jnp.int32, LANES)
        ones = jnp.full((LANES,), 1.0, jnp.float32)
        zeros = jnp.zeros((LANES,), jnp.float32)

        def gat_f(ref, idx):
            return plsc.load_gather(ref, [idx])

        def tri_coords(faces_ref, fidx):
            g0 = plsc.load_gather(faces_ref, [fidx * 3])
            g1 = plsc.load_gather(faces_ref, [fidx * 3 + 1])
            g2 = plsc.load_gather(faces_ref, [fidx * 3 + 2])
            return g0, g1, g2

        def wf_body(i, carry):
            base = i * LANES
            col_l = base + iota
            g0, g1, g2 = tri_coords(of_v, col_l)
            third = jnp.full((LANES,), 1.0 / 3.0, jnp.float32)
            bx = (gat_f(ov_v, g0 * 3) + gat_f(ov_v, g1 * 3) + gat_f(ov_v, g2 * 3)) * third
            by = (gat_f(ov_v, g0 * 3 + 1) + gat_f(ov_v, g1 * 3 + 1) + gat_f(ov_v, g2 * 3 + 1)) * third
            bz = (gat_f(ov_v, g0 * 3 + 2) + gat_f(ov_v, g1 * 3 + 2) + gat_f(ov_v, g2 * 3 + 2)) * third
            col_g = col_l + wid * WF_PER
            nrm = bx * bx + by * by + bz * bz
            nrm = jnp.where(col_g < N_OF, nrm, BIG)
            sl = pl.ds(base, LANES)
            wf_s[0, sl] = -2.0 * bx
            wf_s[1, sl] = -2.0 * by
            wf_s[2, sl] = -2.0 * bz
            wf_s[3, sl] = zeros
            wf_s[4, sl] = zeros
            wf_s[5, sl] = zeros
            wf_s[6, sl] = zeros
            wf_s[7, sl] = zeros
            vnf_s[sl] = nrm
            return carry

        lax.fori_loop(0, WF_PER // LANES, wf_body, 0)

        @pl.when(wid < 8)
        def _qt_fwd():
            def qf_body(i, carry):
                base = i * LANES
                col_g = base + iota + wid * 128
                g0, g1, g2 = tri_coords(sf_v, col_g)
                third = jnp.full((LANES,), 1.0 / 3.0, jnp.float32)
                bx = (gat_f(sv_v, g0 * 3) + gat_f(sv_v, g1 * 3) + gat_f(sv_v, g2 * 3)) * third
                by = (gat_f(sv_v, g0 * 3 + 1) + gat_f(sv_v, g1 * 3 + 1) + gat_f(sv_v, g2 * 3 + 1)) * third
                bz = (gat_f(sv_v, g0 * 3 + 2) + gat_f(sv_v, g1 * 3 + 2) + gat_f(sv_v, g2 * 3 + 2)) * third
                sl = pl.ds(base, LANES)
                qf_s[0, sl] = bx
                qf_s[1, sl] = by
                qf_s[2, sl] = bz
                qf_s[3, sl] = zeros
                qf_s[4, sl] = zeros
                qf_s[5, sl] = zeros
                qf_s[6, sl] = zeros
                qf_s[7, sl] = zeros
                qnf_s[sl] = bx * bx + by * by + bz * bz
                return carry

            lax.fori_loop(0, 128 // LANES, qf_body, 0)
            pltpu.sync_copy(qf_s, qf_h.at[:, pl.ds(wid * 128, 128)])
            pltpu.sync_copy(qnf_s, qnf_h.at[pl.ds(wid * 128, 128)])

        def qr_body(i, carry):
            base = i * LANES
            col_l = base + iota
            col_g = col_l + wid * QR_PER
            f = lax.shift_right_logical(col_g, 3)
            g0, g1, g2 = tri_coords(sf_v, f)
            sl = pl.ds(base, LANES)
            av = a_v[sl]
            bv = b_v[sl]
            cv = c_v[sl]
            sx = av * gat_f(sv_v, g0 * 3) + bv * gat_f(sv_v, g1 * 3) + cv * gat_f(sv_v, g2 * 3)
            sy = av * gat_f(sv_v, g0 * 3 + 1) + bv * gat_f(sv_v, g1 * 3 + 1) + cv * gat_f(sv_v, g2 * 3 + 1)
            sz = av * gat_f(sv_v, g0 * 3 + 2) + bv * gat_f(sv_v, g1 * 3 + 2) + cv * gat_f(sv_v, g2 * 3 + 2)
            qr_s[0, sl] = sx
            qr_s[1, sl] = sy
            qr_s[2, sl] = sz
            qr_s[3, sl] = zeros
            qr_s[4, sl] = zeros
            qr_s[5, sl] = zeros
            qr_s[6, sl] = zeros
            qr_s[7, sl] = zeros
            qnr_s[sl] = sx * sx + sy * sy + sz * sz
            return carry

        lax.fori_loop(0, QR_PER // LANES, qr_body, 0)

        for k in range(3):
            chunk = wid + 32 * k

            @pl.when(chunk < 80)
            def _wr_chunk(chunk=chunk):
                def wr_body(i, carry):
                    base = i * LANES
                    col_g = base + iota + chunk * 128
                    vx = gat_f(ov_v, col_g * 3)
                    vy = gat_f(ov_v, col_g * 3 + 1)
                    vz = gat_f(ov_v, col_g * 3 + 2)
                    nrm = vx * vx + vy * vy + vz * vz
                    nrm = jnp.where(col_g < N_OV, nrm, BIG)
                    sl = pl.ds(base, LANES)
                    wr_s[0, sl] = -2.0 * vx
                    wr_s[1, sl] = -2.0 * vy
                    wr_s[2, sl] = -2.0 * vz
                    wr_s[3, sl] = zeros
                    wr_s[4, sl] = zeros
                    wr_s[5, sl] = zeros
                    wr_s[6, sl] = zeros
                    wr_s[7, sl] = zeros
                    vnr_s[sl] = nrm
                    return carry

                lax.fori_loop(0, 128 // LANES, wr_body, 0)
                pltpu.sync_copy(wr_s, wr_h.at[:, pl.ds(chunk * 128, 128)])
                pltpu.sync_copy(vnr_s, vnr_h.at[pl.ds(chunk * 128, 128)])

        pltpu.sync_copy(wf_s, wf_h.at[:, pl.ds(wid * WF_PER, WF_PER)])
        pltpu.sync_copy(qr_s, qr_h.at[:, pl.ds(wid * QR_PER, QR_PER)])
        pltpu.sync_copy(vnf_s, vnf_h.at[pl.ds(wid * WF_PER, WF_PER)])
        pltpu.sync_copy(qnr_s, qnr_h.at[pl.ds(wid * QR_PER, QR_PER)])

    return body(ov_flat, of_flat, sv_flat, sf_flat, a_flat, b_flat, c_flat)


def _fwd_tc(qh, q2, wh, w2, vn, qn, p_pad):
    jblk = 2048
    njb = P_OF // jblk
    dn = (((0,), (0,)), ((), ()))

    def body(qh_ref, q2_ref, wh_ref, w2_ref, vn_ref, qn_ref, p_ref, fwd_out, sump_out, acc):
        j = pl.program_id(0)
        da = lax.dot_general(qh_ref[...], wh_ref[...], dn, preferred_element_type=jnp.float32)
        db = lax.dot_general(q2_ref[...], w2_ref[...], dn, preferred_element_type=jnp.float32)
        vnr = vn_ref[...]
        m = None
        for k in range(jblk // 128):
            sl = slice(k * 128, (k + 1) * 128)
            chunk = (da[:, sl] + db[:, sl]) + vnr[:, sl]
            m = chunk if m is None else jnp.minimum(m, chunk)
        acc[...] = jnp.where(j == 0, m, jnp.minimum(acc[...], m))

        @pl.when(j == njb - 1)
        def _():
            p = p_ref[...]
            sp = jnp.sum(p)
            row_min = jnp.min(acc[...], axis=1, keepdims=True) + qn_ref[...]
            fwd_out[...] = jnp.full((1, 1), jnp.sum(p * row_min) + 1e-4 * (float(N_SF) - sp), jnp.float32)
            sump_out[...] = jnp.full((1, 1), sp, jnp.float32)

    return pl.pallas_call(
        body,
        grid=(njb,),
        in_specs=[
            pl.BlockSpec((8, P_Q), lambda j: (0, 0)),
            pl.BlockSpec((16, P_Q), lambda j: (0, 0)),
            pl.BlockSpec((8, jblk), lambda j: (0, j)),
            pl.BlockSpec((16, jblk), lambda j: (0, j)),
            pl.BlockSpec((1, jblk), lambda j: (0, j)),
            pl.BlockSpec((P_Q, 1), lambda j: (0, 0)),
            pl.BlockSpec((P_Q, 1), lambda j: (0, 0)),
        ],
        out_specs=[
            pl.BlockSpec((1, 1), lambda j: (0, 0)),
            pl.BlockSpec((1, 1), lambda j: (0, 0)),
        ],
        out_shape=[
            jax.ShapeDtypeStruct((1, 1), jnp.float32),
            jax.ShapeDtypeStruct((1, 1), jnp.float32),
        ],
        scratch_shapes=[pltpu.VMEM((P_Q, 128), jnp.float32)],
    )(qh, q2, wh, w2, vn, qn, p_pad)


def _rev_tc(qh, q2, wh, w2, vn, qn, mask):
    iblk = 1024
    jblk = 2048
    nib = P_SAMP // iblk
    njb = P_OV // jblk
    dn = (((0,), (0,)), ((), ()))

    def body(qh_ref, q2_ref, wh_ref, w2_ref, vn_ref, qn_ref, mask_ref, sum_out, max_out, acc, ssum, smax):
        i = pl.program_id(0)
        j = pl.program_id(1)
        da = lax.dot_general(qh_ref[...], wh_ref[...], dn, preferred_element_type=jnp.float32)
        db = lax.dot_general(q2_ref[...], w2_ref[...], dn, preferred_element_type=jnp.float32)
        vnr = vn_ref[...]
        m = None
        for k in range(jblk // 128):
            sl = slice(k * 128, (k + 1) * 128)
            chunk = (da[:, sl] + db[:, sl]) + vnr[:, sl]
            m = chunk if m is None else jnp.minimum(m, chunk)
        acc[...] = jnp.where(j == 0, m, jnp.minimum(acc[...], m))

        @pl.when(j == njb - 1)
        def _():
            row_min = jnp.min(acc[...], axis=1, keepdims=True) + qn_ref[...]
            mm = mask_ref[...] * row_min
            s = jnp.sum(mm)
            mx = jnp.max(mm)
            ssum[0] = jnp.where(i == 0, s, ssum[0] + s)
            smax[0] = jnp.where(i == 0, mx, jnp.maximum(smax[0], mx))
            sum_out[...] = jnp.full((1, 1), ssum[0], jnp.float32)
            max_out[...] = jnp.full((1, 1), smax[0], jnp.float32)

    return pl.pallas_call(
        body,
        grid=(nib, njb),
        in_specs=[
            pl.BlockSpec((8, iblk), lambda i, j: (0, i)),
            pl.BlockSpec((16, iblk), lambda i, j: (0, i)),
            pl.BlockSpec((8, jblk), lambda i, j: (0, j)),
            pl.BlockSpec((16, jblk), lambda i, j: (0, j)),
            pl.BlockSpec((1, jblk), lambda i, j: (0, j)),
            pl.BlockSpec((iblk, 1), lambda i, j: (i, 0)),
            pl.BlockSpec((iblk, 1), lambda i, j: (i, 0)),
        ],
        out_specs=[
            pl.BlockSpec((1, 1), lambda i, j: (0, 0)),
            pl.BlockSpec((1, 1), lambda i, j: (0, 0)),
        ],
        out_shape=[
            jax.ShapeDtypeStruct((1, 1), jnp.float32),
            jax.ShapeDtypeStruct((1, 1), jnp.float32),
        ],
        scratch_shapes=[
            pltpu.VMEM((iblk, 128), jnp.float32),
            pltpu.SMEM((1,), jnp.float32),
            pltpu.SMEM((1,), jnp.float32),
        ],
    )(qh, q2, wh, w2, vn, qn, mask)


def _tf2x32(k1, k2, x0, x1):
    def rotl(x, d):
        return ((x << _np.uint32(d)) | (x >> _np.uint32(32 - d))).astype(_np.uint32)
    rot = [(13, 15, 26, 6), (17, 29, 16, 24)]
    ks = [_np.uint32(k1), _np.uint32(k2),
          _np.uint32(k1 ^ k2 ^ _np.uint32(0x1BD11BDA))]
    x = [(x0 + ks[0]).astype(_np.uint32), (x1 + ks[1]).astype(_np.uint32)]
    for i, (rs, ka, kb) in enumerate([(rot[0], 1, 2), (rot[1], 2, 0),
                                      (rot[0], 0, 1), (rot[1], 1, 2), (rot[0], 2, 0)]):
        for r in rs:
            x[0] = (x[0] + x[1]).astype(_np.uint32)
            x[1] = rotl(x[1], r)
            x[1] = (x[1] ^ x[0]).astype(_np.uint32)
        x[0] = (x[0] + ks[ka]).astype(_np.uint32)
        x[1] = (x[1] + ks[kb] + _np.uint32(i + 1)).astype(_np.uint32)
    return x[0], x[1]


def _tf_uniform(key, n):
    b1, b2 = _tf2x32(key[0], key[1], _np.zeros(n, _np.uint32),
                     _np.arange(n, dtype=_np.uint32))
    bits = b1 ^ b2
    flt = ((bits >> _np.uint32(9)) | _np.uint32(0x3F800000)).view(_np.float32)
    return _np.maximum(_np.float32(0.0), flt - _np.float32(1.0))


def _sample_consts():
    key = _np.array([0, 42], _np.uint32)
    b1, b2 = _tf2x32(key[0], key[1], _np.zeros(2, _np.uint32),
                     _np.arange(2, dtype=_np.uint32))
    k1, k2 = _np.stack([b1, b2], axis=1)
    r1 = _np.sqrt(_tf_uniform(k1, N_SF * S))
    r2 = _tf_uniform(k2, N_SF * S)
    pad = P_SAMP - N_SF * S
    a = _np.pad((_np.float32(1.0) - r1), (0, pad))
    b = _np.pad((r1 * (_np.float32(1.0) - r2)), (0, pad))
    c = _np.pad((r1 * r2), (0, pad))
    m = (_np.arange(P_SAMP) < N_SF * S).astype(_np.float32).reshape(P_SAMP, 1)
    return a, b, c, m


_A_CONST, _B_CONST, _C_CONST, _MASK_CONST = _sample_consts()


def kernel(original_vertices, original_faces, simplified_vertices,
           simplified_faces, face_probabilities):
    f32 = jnp.float32
    ov_flat = jnp.pad(original_vertices.reshape(-1).astype(f32), (0, 30720 - 3 * N_OV))
    of_flat = jnp.pad(original_faces.reshape(-1).astype(jnp.int32), (0, 3 * P_OF - 3 * N_OF))
    sv_flat = jnp.pad(simplified_vertices.reshape(-1).astype(f32), (0, 2048 - 3 * N_SV))
    sf_flat = jnp.pad(simplified_faces.reshape(-1).astype(jnp.int32), (0, 3072 - 3 * N_SF))

    a_flat, b_flat, c_flat = (jnp.asarray(_A_CONST), jnp.asarray(_B_CONST),
                              jnp.asarray(_C_CONST))

    (w_fwd, qt_fwd, qt_rev, w_rev,
     vn_fwd, qn_fwd, qn_rev, vn_rev) = _sc_pack(
        ov_flat, of_flat, sv_flat, sf_flat, a_flat, b_flat, c_flat)

    p_pad = jnp.pad(face_probabilities.astype(f32), (0, P_Q - N_SF)).reshape(P_Q, 1)
    mask = jnp.asarray(_MASK_CONST)

    def split_hl(x):
        xh = x.astype(jnp.bfloat16)
        xl = (x - xh.astype(f32)).astype(jnp.bfloat16)
        return xh, xl

    qh_f, ql_f = split_hl(qt_fwd)
    wh_f, wl_f = split_hl(w_fwd)
    qh_r, ql_r = split_hl(qt_rev)
    wh_r, wl_r = split_hl(w_rev)
    q2_f = jnp.concatenate([qh_f, ql_f], axis=0)
    w2_f = jnp.concatenate([wl_f, wh_f], axis=0)
    q2_r = jnp.concatenate([qh_r, ql_r], axis=0)
    w2_r = jnp.concatenate([wl_r, wh_r], axis=0)

    fwd_term, sum_p = _fwd_tc(qh_f, q2_f, wh_f, w2_f, vn_fwd.reshape(1, P_OF),
                              qn_fwd.reshape(P_Q, 1), p_pad)
    rev_sum, rev_max = _rev_tc(qh_r, q2_r, wh_r, w2_r, vn_rev.reshape(1, P_OV),
                               qn_rev.reshape(P_SAMP, 1), mask)

    rev_term = 0.1 * sum_p[0, 0] * rev_sum[0, 0] / rev_max[0, 0]
    return fwd_term[0, 0] + rev_term


# speedup vs baseline: 4.0519x; 1.3794x over previous
import functools

import numpy as _np

import jax
import jax.numpy as jnp
from jax import lax
from jax.experimental import pallas as pl
from jax.experimental.pallas import tpu as pltpu
from jax.experimental.pallas import tpu_sc as plsc

N_OV = 10000
N_OF = 20000
N_SV = 600
N_SF = 1000
S = 8

P_OV = 10240
P_OF = 20480
P_Q = 1024
P_SAMP = 8192

NC = 2
NS = 16
NW = NC * NS
LANES = 16

WF_PER = P_OF // NW
QF_PER = P_Q // NW
QR_PER = P_SAMP // NW
WR_PER = P_OV // NW

BIG = 1e30


def _sc_pack(ov_flat, of_flat, sv_flat, sf_flat, a_flat, b_flat, c_flat):
    mesh = plsc.VectorSubcoreMesh(core_axis_name="c", subcore_axis_name="s")

    @functools.partial(
        pl.kernel,
        mesh=mesh,
        compiler_params=pltpu.CompilerParams(needs_layout_passes=False),
        out_type=[
            jax.ShapeDtypeStruct((8, P_OF), jnp.float32),
            jax.ShapeDtypeStruct((8, P_Q), jnp.float32),
            jax.ShapeDtypeStruct((8, P_SAMP), jnp.float32),
            jax.ShapeDtypeStruct((8, P_OV), jnp.float32),
            jax.ShapeDtypeStruct((P_OF,), jnp.float32),
            jax.ShapeDtypeStruct((P_Q,), jnp.float32),
            jax.ShapeDtypeStruct((P_SAMP,), jnp.float32),
            jax.ShapeDtypeStruct((P_OV,), jnp.float32),
        ],
        scratch_types=[
            pltpu.VMEM((30720,), jnp.float32),
            pltpu.VMEM((3 * WF_PER,), jnp.int32),
            pltpu.VMEM((2048,), jnp.float32),
            pltpu.VMEM((3072,), jnp.int32),
            pltpu.VMEM((QR_PER,), jnp.float32),
            pltpu.VMEM((QR_PER,), jnp.float32),
            pltpu.VMEM((QR_PER,), jnp.float32),
            pltpu.VMEM((8, WF_PER), jnp.float32),
            pltpu.VMEM((8, 128), jnp.float32),
            pltpu.VMEM((8, QR_PER), jnp.float32),
            pltpu.VMEM((8, 128), jnp.float32),
            pltpu.VMEM((WF_PER,), jnp.float32),
            pltpu.VMEM((128,), jnp.float32),
            pltpu.VMEM((QR_PER,), jnp.float32),
            pltpu.VMEM((128,), jnp.float32),
        ],
    )
    def body(ov_h, of_h, sv_h, sf_h, a_h, b_h, c_h,
             wf_h, qf_h, qr_h, wr_h, vnf_h, qnf_h, qnr_h, vnr_h,
             ov_v, of_v, sv_v, sf_v, a_v, b_v, c_v,
             wf_s, qf_s, qr_s, wr_s, vnf_s, qnf_s, qnr_s, vnr_s):
        wid = lax.axis_index("s") * NC + lax.axis_index("c")

        pltpu.sync_copy(ov_h, ov_v)
        pltpu.sync_copy(of_h.at[pl.ds(wid * 3 * WF_PER, 3 * WF_PER)], of_v)
        pltpu.sync_copy(sv_h, sv_v)
        pltpu.sync_copy(sf_h, sf_v)
        pltpu.sync_copy(a_h.at[pl.ds(wid * QR_PER, QR_PER)], a_v)
        pltpu.sync_copy(b_h.at[pl.ds(wid * QR_PER, QR_PER)], b_v)
        pltpu.sync_copy(c_h.at[pl.ds(wid * QR_PER, QR_PER)], c_v)

        iota = lax.iota(jnp.int32, LANES)
        ones = jnp.full((LANES,), 1.0, jnp.float32)
        zeros = jnp.zeros((LANES,), jnp.float32)

        def gat_f(ref, idx):
            return plsc.load_gather(ref, [idx])

        def tri_coords(faces_ref, fidx):
            g0 = plsc.load_gather(faces_ref, [fidx * 3])
            g1 = plsc.load_gather(faces_ref, [fidx * 3 + 1])
            g2 = plsc.load_gather(faces_ref, [fidx * 3 + 2])
            return g0, g1, g2

        def wf_body(i, carry):
            base = i * LANES
            col_l = base + iota
            g0, g1, g2 = tri_coords(of_v, col_l)
            third = jnp.full((LANES,), 1.0 / 3.0, jnp.float32)
            bx = (gat_f(ov_v, g0 * 3) + gat_f(ov_v, g1 * 3) + gat_f(ov_v, g2 * 3)) * third
            by = (gat_f(ov_v, g0 * 3 + 1) + gat_f(ov_v, g1 * 3 + 1) + gat_f(ov_v, g2 * 3 + 1)) * third
            bz = (gat_f(ov_v, g0 * 3 + 2) + gat_f(ov_v, g1 * 3 + 2) + gat_f(ov_v, g2 * 3 + 2)) * third
            col_g = col_l + wid * WF_PER
            nrm = bx * bx + by * by + bz * bz
            nrm = jnp.where(col_g < N_OF, nrm, BIG)
            sl = pl.ds(base, LANES)
            wf_s[0, sl] = -2.0 * bx
            wf_s[1, sl] = -2.0 * by
            wf_s[2, sl] = -2.0 * bz
            wf_s[3, sl] = zeros
            wf_s[4, sl] = zeros
            wf_s[5, sl] = zeros
            wf_s[6, sl] = zeros
            wf_s[7, sl] = zeros
            vnf_s[sl] = nrm
            return carry

        lax.fori_loop(0, WF_PER // LANES, wf_body, 0)

        @pl.when(wid < 8)
        def _qt_fwd():
            def qf_body(i, carry):
                base = i * LANES
                col_g = base + iota + wid * 128
                g0, g1, g2 = tri_coords(sf_v, col_g)
                third = jnp.full((LANES,), 1.0 / 3.0, jnp.float32)
                bx = (gat_f(sv_v, g0 * 3) + gat_f(sv_v, g1 * 3) + gat_f(sv_v, g2 * 3)) * third
                by = (gat_f(sv_v, g0 * 3 + 1) + gat_f(sv_v, g1 * 3 + 1) + gat_f(sv_v, g2 * 3 + 1)) * third
                bz = (gat_f(sv_v, g0 * 3 + 2) + gat_f(sv_v, g1 * 3 + 2) + gat_f(sv_v, g2 * 3 + 2)) * third
                sl = pl.ds(base, LANES)
                qf_s[0, sl] = bx
                qf_s[1, sl] = by
                qf_s[2, sl] = bz
                qf_s[3, sl] = zeros
                qf_s[4, sl] = zeros
                qf_s[5, sl] = zeros
                qf_s[6, sl] = zeros
                qf_s[7, sl] = zeros
                qnf_s[sl] = bx * bx + by * by + bz * bz
                return carry

            lax.fori_loop(0, 128 // LANES, qf_body, 0)
            pltpu.sync_copy(qf_s, qf_h.at[:, pl.ds(wid * 128, 128)])
            pltpu.sync_copy(qnf_s, qnf_h.at[pl.ds(wid * 128, 128)])

        def qr_body(i, carry):
            base = i * LANES
            col_l = base + iota
            col_g = col_l + wid * QR_PER
            f = lax.shift_right_logical(col_g, 3)
            g0, g1, g2 = tri_coords(sf_v, f)
            sl = pl.ds(base, LANES)
            av = a_v[sl]
            bv = b_v[sl]
            cv = c_v[sl]
            sx = av * gat_f(sv_v, g0 * 3) + bv * gat_f(sv_v, g1 * 3) + cv * gat_f(sv_v, g2 * 3)
            sy = av * gat_f(sv_v, g0 * 3 + 1) + bv * gat_f(sv_v, g1 * 3 + 1) + cv * gat_f(sv_v, g2 * 3 + 1)
            sz = av * gat_f(sv_v, g0 * 3 + 2) + bv * gat_f(sv_v, g1 * 3 + 2) + cv * gat_f(sv_v, g2 * 3 + 2)
            qr_s[0, sl] = sx
            qr_s[1, sl] = sy
            qr_s[2, sl] = sz
            qr_s[3, sl] = zeros
            qr_s[4, sl] = zeros
            qr_s[5, sl] = zeros
            qr_s[6, sl] = zeros
            qr_s[7, sl] = zeros
            qnr_s[sl] = sx * sx + sy * sy + sz * sz
            return carry

        lax.fori_loop(0, QR_PER // LANES, qr_body, 0)

        for k in range(3):
            chunk = wid + 32 * k

            @pl.when(chunk < 80)
            def _wr_chunk(chunk=chunk):
                def wr_body(i, carry):
                    base = i * LANES
                    col_g = base + iota + chunk * 128
                    vx = gat_f(ov_v, col_g * 3)
                    vy = gat_f(ov_v, col_g * 3 + 1)
                    vz = gat_f(ov_v, col_g * 3 + 2)
                    nrm = vx * vx + vy * vy + vz * vz
                    nrm = jnp.where(col_g < N_OV, nrm, BIG)
                    sl = pl.ds(base, LANES)
                    wr_s[0, sl] = -2.0 * vx
                    wr_s[1, sl] = -2.0 * vy
                    wr_s[2, sl] = -2.0 * vz
                    wr_s[3, sl] = zeros
                    wr_s[4, sl] = zeros
                    wr_s[5, sl] = zeros
                    wr_s[6, sl] = zeros
                    wr_s[7, sl] = zeros
                    vnr_s[sl] = nrm
                    return carry

                lax.fori_loop(0, 128 // LANES, wr_body, 0)
                pltpu.sync_copy(wr_s, wr_h.at[:, pl.ds(chunk * 128, 128)])
                pltpu.sync_copy(vnr_s, vnr_h.at[pl.ds(chunk * 128, 128)])

        pltpu.sync_copy(wf_s, wf_h.at[:, pl.ds(wid * WF_PER, WF_PER)])
        pltpu.sync_copy(qr_s, qr_h.at[:, pl.ds(wid * QR_PER, QR_PER)])
        pltpu.sync_copy(vnf_s, vnf_h.at[pl.ds(wid * WF_PER, WF_PER)])
        pltpu.sync_copy(qnr_s, qnr_h.at[pl.ds(wid * QR_PER, QR_PER)])

    return body(ov_flat, of_flat, sv_flat, sf_flat, a_flat, b_flat, c_flat)


def _fwd_tc(q3, w3, vn, qn, p_pad):
    jblk = 2048
    njb = P_OF // jblk
    dn = (((0,), (0,)), ((), ()))

    def body(q3_ref, w3_ref, vn_ref, qn_ref, p_ref, fwd_out, sump_out, acc):
        j = pl.program_id(0)
        d = lax.dot_general(q3_ref[...], w3_ref[...], dn, preferred_element_type=jnp.float32)
        vnr = vn_ref[...]
        m = None
        for k in range(jblk // 128):
            sl = slice(k * 128, (k + 1) * 128)
            chunk = d[:, sl] + vnr[:, sl]
            m = chunk if m is None else jnp.minimum(m, chunk)
        acc[...] = jnp.where(j == 0, m, jnp.minimum(acc[...], m))

        @pl.when(j == njb - 1)
        def _():
            p = p_ref[...]
            sp = jnp.sum(p)
            row_min = jnp.min(acc[...], axis=1, keepdims=True) + qn_ref[...]
            fwd_out[...] = jnp.full((1, 1), jnp.sum(p * row_min) + 1e-4 * (float(N_SF) - sp), jnp.float32)
            sump_out[...] = jnp.full((1, 1), sp, jnp.float32)

    return pl.pallas_call(
        body,
        grid=(njb,),
        in_specs=[
            pl.BlockSpec((32, P_Q), lambda j: (0, 0)),
            pl.BlockSpec((32, jblk), lambda j: (0, j)),
            pl.BlockSpec((1, jblk), lambda j: (0, j)),
            pl.BlockSpec((P_Q, 1), lambda j: (0, 0)),
            pl.BlockSpec((P_Q, 1), lambda j: (0, 0)),
        ],
        out_specs=[
            pl.BlockSpec((1, 1), lambda j: (0, 0)),
            pl.BlockSpec((1, 1), lambda j: (0, 0)),
        ],
        out_shape=[
            jax.ShapeDtypeStruct((1, 1), jnp.float32),
            jax.ShapeDtypeStruct((1, 1), jnp.float32),
        ],
        scratch_shapes=[pltpu.VMEM((P_Q, 128), jnp.float32)],
    )(q3, w3, vn, qn, p_pad)


def _rev_tc(q3, w3, vn, qn, mask):
    iblk = 1024
    jblk = 2048
    nib = P_SAMP // iblk
    njb = P_OV // jblk
    dn = (((0,), (0,)), ((), ()))

    def body(q3_ref, w3_ref, vn_ref, qn_ref, mask_ref, sum_out, max_out, acc, ssum, smax):
        i = pl.program_id(0)
        j = pl.program_id(1)
        d = lax.dot_general(q3_ref[...], w3_ref[...], dn, preferred_element_type=jnp.float32)
        vnr = vn_ref[...]
        m = None
        for k in range(jblk // 128):
            sl = slice(k * 128, (k + 1) * 128)
            chunk = d[:, sl] + vnr[:, sl]
            m = chunk if m is None else jnp.minimum(m, chunk)
        acc[...] = jnp.where(j == 0, m, jnp.minimum(acc[...], m))

        @pl.when(j == njb - 1)
        def _():
            row_min = jnp.min(acc[...], axis=1, keepdims=True) + qn_ref[...]
            mm = mask_ref[...] * row_min
            s = jnp.sum(mm)
            mx = jnp.max(mm)
            ssum[0] = jnp.where(i == 0, s, ssum[0] + s)
            smax[0] = jnp.where(i == 0, mx, jnp.maximum(smax[0], mx))
            sum_out[...] = jnp.full((1, 1), ssum[0], jnp.float32)
            max_out[...] = jnp.full((1, 1), smax[0], jnp.float32)

    return pl.pallas_call(
        body,
        grid=(nib, njb),
        in_specs=[
            pl.BlockSpec((32, iblk), lambda i, j: (0, i)),
            pl.BlockSpec((32, jblk), lambda i, j: (0, j)),
            pl.BlockSpec((1, jblk), lambda i, j: (0, j)),
            pl.BlockSpec((iblk, 1), lambda i, j: (i, 0)),
            pl.BlockSpec((iblk, 1), lambda i, j: (i, 0)),
        ],
        out_specs=[
            pl.BlockSpec((1, 1), lambda i, j: (0, 0)),
            pl.BlockSpec((1, 1), lambda i, j: (0, 0)),
        ],
        out_shape=[
            jax.ShapeDtypeStruct((1, 1), jnp.float32),
            jax.ShapeDtypeStruct((1, 1), jnp.float32),
        ],
        scratch_shapes=[
            pltpu.VMEM((iblk, 128), jnp.float32),
            pltpu.SMEM((1,), jnp.float32),
            pltpu.SMEM((1,), jnp.float32),
        ],
    )(q3, w3, vn, qn, mask)


def _tf2x32(k1, k2, x0, x1):
    def rotl(x, d):
        return ((x << _np.uint32(d)) | (x >> _np.uint32(32 - d))).astype(_np.uint32)
    rot = [(13, 15, 26, 6), (17, 29, 16, 24)]
    ks = [_np.uint32(k1), _np.uint32(k2),
          _np.uint32(k1 ^ k2 ^ _np.uint32(0x1BD11BDA))]
    x = [(x0 + ks[0]).astype(_np.uint32), (x1 + ks[1]).astype(_np.uint32)]
    for i, (rs, ka, kb) in enumerate([(rot[0], 1, 2), (rot[1], 2, 0),
                                      (rot[0], 0, 1), (rot[1], 1, 2), (rot[0], 2, 0)]):
        for r in rs:
            x[0] = (x[0] + x[1]).astype(_np.uint32)
            x[1] = rotl(x[1], r)
            x[1] = (x[1] ^ x[0]).astype(_np.uint32)
        x[0] = (x[0] + ks[ka]).astype(_np.uint32)
        x[1] = (x[1] + ks[kb] + _np.uint32(i + 1)).astype(_np.uint32)
    return x[0], x[1]


def _tf_uniform(key, n):
    b1, b2 = _tf2x32(key[0], key[1], _np.zeros(n, _np.uint32),
                     _np.arange(n, dtype=_np.uint32))
    bits = b1 ^ b2
    flt = ((bits >> _np.uint32(9)) | _np.uint32(0x3F800000)).view(_np.float32)
    return _np.maximum(_np.float32(0.0), flt - _np.float32(1.0))


def _sample_consts():
    key = _np.array([0, 42], _np.uint32)
    b1, b2 = _tf2x32(key[0], key[1], _np.zeros(2, _np.uint32),
                     _np.arange(2, dtype=_np.uint32))
    k1, k2 = _np.stack([b1, b2], axis=1)
    r1 = _np.sqrt(_tf_uniform(k1, N_SF * S))
    r2 = _tf_uniform(k2, N_SF * S)
    pad = P_SAMP - N_SF * S
    a = _np.pad((_np.float32(1.0) - r1), (0, pad))
    b = _np.pad((r1 * (_np.float32(1.0) - r2)), (0, pad))
    c = _np.pad((r1 * r2), (0, pad))
    m = (_np.arange(P_SAMP) < N_SF * S).astype(_np.float32).reshape(P_SAMP, 1)
    return a, b, c, m


_A_CONST, _B_CONST, _C_CONST, _MASK_CONST = _sample_consts()


def kernel(original_vertices, original_faces, simplified_vertices,
           simplified_faces, face_probabilities):
    f32 = jnp.float32
    ov_flat = jnp.pad(original_vertices.reshape(-1).astype(f32), (0, 30720 - 3 * N_OV))
    of_flat = jnp.pad(original_faces.reshape(-1).astype(jnp.int32), (0, 3 * P_OF - 3 * N_OF))
    sv_flat = jnp.pad(simplified_vertices.reshape(-1).astype(f32), (0, 2048 - 3 * N_SV))
    sf_flat = jnp.pad(simplified_faces.reshape(-1).astype(jnp.int32), (0, 3072 - 3 * N_SF))

    a_flat, b_flat, c_flat = (jnp.asarray(_A_CONST), jnp.asarray(_B_CONST),
                              jnp.asarray(_C_CONST))

    (w_fwd, qt_fwd, qt_rev, w_rev,
     vn_fwd, qn_fwd, qn_rev, vn_rev) = _sc_pack(
        ov_flat, of_flat, sv_flat, sf_flat, a_flat, b_flat, c_flat)

    p_pad = jnp.pad(face_probabilities.astype(f32), (0, P_Q - N_SF)).reshape(P_Q, 1)
    mask = jnp.asarray(_MASK_CONST)

    def split_hl(x):
        xh = x.astype(jnp.bfloat16)
        xl = (x - xh.astype(f32)).astype(jnp.bfloat16)
        return xh, xl

    def stack3(q, w):
        qh, ql = split_hl(q)
        wh, wl = split_hl(w)
        zq = jnp.zeros_like(qh)
        zw = jnp.zeros_like(wh)
        q3 = jnp.concatenate([qh, qh, ql, zq], axis=0)
        w3 = jnp.concatenate([wh, wl, wh, zw], axis=0)
        return q3, w3

    q3_f, w3_f = stack3(qt_fwd, w_fwd)
    q3_r, w3_r = stack3(qt_rev, w_rev)

    fwd_term, sum_p = _fwd_tc(q3_f, w3_f, vn_fwd.reshape(1, P_OF),
                              qn_fwd.reshape(P_Q, 1), p_pad)
    rev_sum, rev_max = _rev_tc(q3_r, w3_r, vn_rev.reshape(1, P_OV),
                               qn_rev.reshape(P_SAMP, 1), mask)

    rev_term = 0.1 * sum_p[0, 0] * rev_sum[0, 0] / rev_max[0, 0]
    return fwd_term[0, 0] + rev_term


# speedup vs baseline: 4.2144x; 1.0401x over previous
import functools

import numpy as _np

import jax
import jax.numpy as jnp
from jax import lax
from jax.experimental import pallas as pl
from jax.experimental.pallas import tpu as pltpu
from jax.experimental.pallas import tpu_sc as plsc

N_OV = 10000
N_OF = 20000
N_SV = 600
N_SF = 1000
S = 8

P_OV = 10240
P_OF = 20480
P_Q = 1024
P_SAMP = 8192

NC = 2
NS = 16
NW = NC * NS
LANES = 16

WF_PER = P_OF // NW
QF_PER = P_Q // NW
QR_PER = P_SAMP // NW
WR_PER = P_OV // NW

BIG = 1e30


def _sc_pack(ov_flat, of_flat, sv_flat, sf_flat, a_flat, b_flat, c_flat):
    mesh = plsc.VectorSubcoreMesh(core_axis_name="c", subcore_axis_name="s")

    @functools.partial(
        pl.kernel,
        mesh=mesh,
        compiler_params=pltpu.CompilerParams(needs_layout_passes=False),
        out_type=[
            jax.ShapeDtypeStruct((8, P_OF), jnp.float32),
            jax.ShapeDtypeStruct((8, P_Q), jnp.float32),
            jax.ShapeDtypeStruct((8, P_SAMP), jnp.float32),
            jax.ShapeDtypeStruct((8, P_OV), jnp.float32),
            jax.ShapeDtypeStruct((P_OF,), jnp.float32),
            jax.ShapeDtypeStruct((P_Q,), jnp.float32),
            jax.ShapeDtypeStruct((P_SAMP,), jnp.float32),
            jax.ShapeDtypeStruct((P_OV,), jnp.float32),
        ],
        scratch_types=[
            pltpu.VMEM((30720,), jnp.float32),
            pltpu.VMEM((3 * WF_PER,), jnp.int32),
            pltpu.VMEM((2048,), jnp.float32),
            pltpu.VMEM((3072,), jnp.int32),
            pltpu.VMEM((QR_PER,), jnp.float32),
            pltpu.VMEM((QR_PER,), jnp.float32),
            pltpu.VMEM((QR_PER,), jnp.float32),
            pltpu.VMEM((8, WF_PER), jnp.float32),
            pltpu.VMEM((8, 128), jnp.float32),
            pltpu.VMEM((8, QR_PER), jnp.float32),
            pltpu.VMEM((8, 128), jnp.float32),
            pltpu.VMEM((WF_PER,), jnp.float32),
            pltpu.VMEM((128,), jnp.float32),
            pltpu.VMEM((QR_PER,), jnp.float32),
            pltpu.VMEM((128,), jnp.float32),
        ],
    )
    def body(ov_h, of_h, sv_h, sf_h, a_h, b_h, c_h,
             wf_h, qf_h, qr_h, wr_h, vnf_h, qnf_h, qnr_h, vnr_h,
             ov_v, of_v, sv_v, sf_v, a_v, b_v, c_v,
             wf_s, qf_s, qr_s, wr_s, vnf_s, qnf_s, qnr_s, vnr_s):
        wid = lax.axis_index("s") * NC + lax.axis_index("c")

        pltpu.sync_copy(ov_h, ov_v)
        pltpu.sync_copy(of_h.at[pl.ds(wid * 3 * WF_PER, 3 * WF_PER)], of_v)
        pltpu.sync_copy(sv_h, sv_v)
        pltpu.sync_copy(sf_h, sf_v)
        pltpu.sync_copy(a_h.at[pl.ds(wid * QR_PER, QR_PER)], a_v)
        pltpu.sync_copy(b_h.at[pl.ds(wid * QR_PER, QR_PER)], b_v)
        pltpu.sync_copy(c_h.at[pl.ds(wid * QR_PER, QR_PER)], c_v)

        iota = lax.iota(jnp.int32, LANES)
        ones = jnp.full((LANES,), 1.0, jnp.float32)
        zeros = jnp.zeros((LANES,), jnp.float32)

        def gat_f(ref, idx):
            return plsc.load_gather(ref, [idx])

        def tri_coords(faces_ref, fidx):
            g0 = plsc.load_gather(faces_ref, [fidx * 3])
            g1 = plsc.load_gather(faces_ref, [fidx * 3 + 1])
            g2 = plsc.load_gather(faces_ref, [fidx * 3 + 2])
            return g0, g1, g2

        def wf_body(i, carry):
            base = i * LANES
            col_l = base + iota
            g0, g1, g2 = tri_coords(of_v, col_l)
            third = jnp.full((LANES,), 1.0 / 3.0, jnp.float32)
            bx = (gat_f(ov_v, g0 * 3) + gat_f(ov_v, g1 * 3) + gat_f(ov_v, g2 * 3)) * third
            by = (gat_f(ov_v, g0 * 3 + 1) + gat_f(ov_v, g1 * 3 + 1) + gat_f(ov_v, g2 * 3 + 1)) * third
            bz = (gat_f(ov_v, g0 * 3 + 2) + gat_f(ov_v, g1 * 3 + 2) + gat_f(ov_v, g2 * 3 + 2)) * third
            col_g = col_l + wid * WF_PER
            nrm = bx * bx + by * by + bz * bz
            nrm = jnp.where(col_g < N_OF, nrm, BIG)
            sl = pl.ds(base, LANES)
            wf_s[0, sl] = -2.0 * bx
            wf_s[1, sl] = -2.0 * by
            wf_s[2, sl] = -2.0 * bz
            wf_s[3, sl] = zeros
            wf_s[4, sl] = zeros
            wf_s[5, sl] = zeros
            wf_s[6, sl] = zeros
            wf_s[7, sl] = zeros
            vnf_s[sl] = nrm
            return carry

        lax.fori_loop(0, WF_PER // LANES, wf_body, 0)

        @pl.when(wid < 8)
        def _qt_fwd():
            def qf_body(i, carry):
                base = i * LANES
                col_g = base + iota + wid * 128
                g0, g1, g2 = tri_coords(sf_v, col_g)
                third = jnp.full((LANES,), 1.0 / 3.0, jnp.float32)
                bx = (gat_f(sv_v, g0 * 3) + gat_f(sv_v, g1 * 3) + gat_f(sv_v, g2 * 3)) * third
                by = (gat_f(sv_v, g0 * 3 + 1) + gat_f(sv_v, g1 * 3 + 1) + gat_f(sv_v, g2 * 3 + 1)) * third
                bz = (gat_f(sv_v, g0 * 3 + 2) + gat_f(sv_v, g1 * 3 + 2) + gat_f(sv_v, g2 * 3 + 2)) * third
                sl = pl.ds(base, LANES)
                qf_s[0, sl] = bx
                qf_s[1, sl] = by
                qf_s[2, sl] = bz
                qf_s[3, sl] = zeros
                qf_s[4, sl] = zeros
                qf_s[5, sl] = zeros
                qf_s[6, sl] = zeros
                qf_s[7, sl] = zeros
                qnf_s[sl] = bx * bx + by * by + bz * bz
                return carry

            lax.fori_loop(0, 128 // LANES, qf_body, 0)
            pltpu.sync_copy(qf_s, qf_h.at[:, pl.ds(wid * 128, 128)])
            pltpu.sync_copy(qnf_s, qnf_h.at[pl.ds(wid * 128, 128)])

        def qr_body(i, carry):
            base = i * LANES
            col_l = base + iota
            col_g = col_l + wid * QR_PER
            f = lax.shift_right_logical(col_g, 3)
            g0, g1, g2 = tri_coords(sf_v, f)
            sl = pl.ds(base, LANES)
            av = a_v[sl]
            bv = b_v[sl]
            cv = c_v[sl]
            sx = av * gat_f(sv_v, g0 * 3) + bv * gat_f(sv_v, g1 * 3) + cv * gat_f(sv_v, g2 * 3)
            sy = av * gat_f(sv_v, g0 * 3 + 1) + bv * gat_f(sv_v, g1 * 3 + 1) + cv * gat_f(sv_v, g2 * 3 + 1)
            sz = av * gat_f(sv_v, g0 * 3 + 2) + bv * gat_f(sv_v, g1 * 3 + 2) + cv * gat_f(sv_v, g2 * 3 + 2)
            qr_s[0, sl] = sx
            qr_s[1, sl] = sy
            qr_s[2, sl] = sz
            qr_s[3, sl] = zeros
            qr_s[4, sl] = zeros
            qr_s[5, sl] = zeros
            qr_s[6, sl] = zeros
            qr_s[7, sl] = zeros
            qnr_s[sl] = sx * sx + sy * sy + sz * sz
            return carry

        lax.fori_loop(0, QR_PER // LANES, qr_body, 0)

        for k in range(3):
            chunk = wid + 32 * k

            @pl.when(chunk < 80)
            def _wr_chunk(chunk=chunk):
                def wr_body(i, carry):
                    base = i * LANES
                    col_g = base + iota + chunk * 128
                    vx = gat_f(ov_v, col_g * 3)
                    vy = gat_f(ov_v, col_g * 3 + 1)
                    vz = gat_f(ov_v, col_g * 3 + 2)
                    nrm = vx * vx + vy * vy + vz * vz
                    nrm = jnp.where(col_g < N_OV, nrm, BIG)
                    sl = pl.ds(base, LANES)
                    wr_s[0, sl] = -2.0 * vx
                    wr_s[1, sl] = -2.0 * vy
                    wr_s[2, sl] = -2.0 * vz
                    wr_s[3, sl] = zeros
                    wr_s[4, sl] = zeros
                    wr_s[5, sl] = zeros
                    wr_s[6, sl] = zeros
                    wr_s[7, sl] = zeros
                    vnr_s[sl] = nrm
                    return carry

                lax.fori_loop(0, 128 // LANES, wr_body, 0)
                pltpu.sync_copy(wr_s, wr_h.at[:, pl.ds(chunk * 128, 128)])
                pltpu.sync_copy(vnr_s, vnr_h.at[pl.ds(chunk * 128, 128)])

        pltpu.sync_copy(wf_s, wf_h.at[:, pl.ds(wid * WF_PER, WF_PER)])
        pltpu.sync_copy(qr_s, qr_h.at[:, pl.ds(wid * QR_PER, QR_PER)])
        pltpu.sync_copy(vnf_s, vnf_h.at[pl.ds(wid * WF_PER, WF_PER)])
        pltpu.sync_copy(qnr_s, qnr_h.at[pl.ds(wid * QR_PER, QR_PER)])

    return body(ov_flat, of_flat, sv_flat, sf_flat, a_flat, b_flat, c_flat)


def _stack3_in_kernel(x):
    xh = x.astype(jnp.bfloat16)
    xl = (x - xh.astype(jnp.float32)).astype(jnp.bfloat16)
    return xh, xl


def _fwd_tc(qt, w, vn, qn, p_pad):
    jblk = 2048
    njb = P_OF // jblk
    dn = (((0,), (0,)), ((), ()))

    def body(qt_ref, w_ref, vn_ref, qn_ref, p_ref, fwd_out, sump_out, acc):
        j = pl.program_id(0)
        qh, ql = _stack3_in_kernel(qt_ref[...])
        wh, wl = _stack3_in_kernel(w_ref[...])
        q3 = jnp.concatenate([qh, qh, ql], axis=0)
        w3 = jnp.concatenate([wh, wl, wh], axis=0)
        d = lax.dot_general(q3, w3, dn, preferred_element_type=jnp.float32)
        vnr = vn_ref[...]
        m = None
        for k in range(jblk // 128):
            sl = slice(k * 128, (k + 1) * 128)
            chunk = d[:, sl] + vnr[:, sl]
            m = chunk if m is None else jnp.minimum(m, chunk)
        acc[...] = jnp.where(j == 0, m, jnp.minimum(acc[...], m))

        @pl.when(j == njb - 1)
        def _():
            p = p_ref[...]
            sp = jnp.sum(p)
            row_min = jnp.min(acc[...], axis=1, keepdims=True) + qn_ref[...]
            fwd_out[...] = jnp.full((1, 1), jnp.sum(p * row_min) + 1e-4 * (float(N_SF) - sp), jnp.float32)
            sump_out[...] = jnp.full((1, 1), sp, jnp.float32)

    return pl.pallas_call(
        body,
        grid=(njb,),
        in_specs=[
            pl.BlockSpec((8, P_Q), lambda j: (0, 0)),
            pl.BlockSpec((8, jblk), lambda j: (0, j)),
            pl.BlockSpec((1, jblk), lambda j: (0, j)),
            pl.BlockSpec((P_Q, 1), lambda j: (0, 0)),
            pl.BlockSpec((P_Q, 1), lambda j: (0, 0)),
        ],
        out_specs=[
            pl.BlockSpec((1, 1), lambda j: (0, 0)),
            pl.BlockSpec((1, 1), lambda j: (0, 0)),
        ],
        out_shape=[
            jax.ShapeDtypeStruct((1, 1), jnp.float32),
            jax.ShapeDtypeStruct((1, 1), jnp.float32),
        ],
        scratch_shapes=[pltpu.VMEM((P_Q, 128), jnp.float32)],
    )(qt, w, vn, qn, p_pad)


def _rev_tc(qt, w, vn, qn, mask):
    iblk = 1024
    jblk = 2048
    nib = P_SAMP // iblk
    njb = P_OV // jblk
    dn = (((0,), (0,)), ((), ()))

    def body(qt_ref, w_ref, vn_ref, qn_ref, mask_ref, sum_out, max_out, acc, ssum, smax):
        i = pl.program_id(0)
        j = pl.program_id(1)
        qh, ql = _stack3_in_kernel(qt_ref[...])
        wh, wl = _stack3_in_kernel(w_ref[...])
        q3 = jnp.concatenate([qh, qh, ql], axis=0)
        w3 = jnp.concatenate([wh, wl, wh], axis=0)
        d = lax.dot_general(q3, w3, dn, preferred_element_type=jnp.float32)
        vnr = vn_ref[...]
        m = None
        for k in range(jblk // 128):
            sl = slice(k * 128, (k + 1) * 128)
            chunk = d[:, sl] + vnr[:, sl]
            m = chunk if m is None else jnp.minimum(m, chunk)
        acc[...] = jnp.where(j == 0, m, jnp.minimum(acc[...], m))

        @pl.when(j == njb - 1)
        def _():
            row_min = jnp.min(acc[...], axis=1, keepdims=True) + qn_ref[...]
            mm = mask_ref[...] * row_min
            s = jnp.sum(mm)
            mx = jnp.max(mm)
            ssum[0] = jnp.where(i == 0, s, ssum[0] + s)
            smax[0] = jnp.where(i == 0, mx, jnp.maximum(smax[0], mx))
            sum_out[...] = jnp.full((1, 1), ssum[0], jnp.float32)
            max_out[...] = jnp.full((1, 1), smax[0], jnp.float32)

    return pl.pallas_call(
        body,
        grid=(nib, njb),
        in_specs=[
            pl.BlockSpec((8, iblk), lambda i, j: (0, i)),
            pl.BlockSpec((8, jblk), lambda i, j: (0, j)),
            pl.BlockSpec((1, jblk), lambda i, j: (0, j)),
            pl.BlockSpec((iblk, 1), lambda i, j: (i, 0)),
            pl.BlockSpec((iblk, 1), lambda i, j: (i, 0)),
        ],
        out_specs=[
            pl.BlockSpec((1, 1), lambda i, j: (0, 0)),
            pl.BlockSpec((1, 1), lambda i, j: (0, 0)),
        ],
        out_shape=[
            jax.ShapeDtypeStruct((1, 1), jnp.float32),
            jax.ShapeDtypeStruct((1, 1), jnp.float32),
        ],
        scratch_shapes=[
            pltpu.VMEM((iblk, 128), jnp.float32),
            pltpu.SMEM((1,), jnp.float32),
            pltpu.SMEM((1,), jnp.float32),
        ],
    )(qt, w, vn, qn, mask)


def _tf2x32(k1, k2, x0, x1):
    def rotl(x, d):
        return ((x << _np.uint32(d)) | (x >> _np.uint32(32 - d))).astype(_np.uint32)
    rot = [(13, 15, 26, 6), (17, 29, 16, 24)]
    ks = [_np.uint32(k1), _np.uint32(k2),
          _np.uint32(k1 ^ k2 ^ _np.uint32(0x1BD11BDA))]
    x = [(x0 + ks[0]).astype(_np.uint32), (x1 + ks[1]).astype(_np.uint32)]
    for i, (rs, ka, kb) in enumerate([(rot[0], 1, 2), (rot[1], 2, 0),
                                      (rot[0], 0, 1), (rot[1], 1, 2), (rot[0], 2, 0)]):
        for r in rs:
            x[0] = (x[0] + x[1]).astype(_np.uint32)
            x[1] = rotl(x[1], r)
            x[1] = (x[1] ^ x[0]).astype(_np.uint32)
        x[0] = (x[0] + ks[ka]).astype(_np.uint32)
        x[1] = (x[1] + ks[kb] + _np.uint32(i + 1)).astype(_np.uint32)
    return x[0], x[1]


def _tf_uniform(key, n):
    b1, b2 = _tf2x32(key[0], key[1], _np.zeros(n, _np.uint32),
                     _np.arange(n, dtype=_np.uint32))
    bits = b1 ^ b2
    flt = ((bits >> _np.uint32(9)) | _np.uint32(0x3F800000)).view(_np.float32)
    return _np.maximum(_np.float32(0.0), flt - _np.float32(1.0))


def _sample_consts():
    key = _np.array([0, 42], _np.uint32)
    b1, b2 = _tf2x32(key[0], key[1], _np.zeros(2, _np.uint32),
                     _np.arange(2, dtype=_np.uint32))
    k1, k2 = _np.stack([b1, b2], axis=1)
    r1 = _np.sqrt(_tf_uniform(k1, N_SF * S))
    r2 = _tf_uniform(k2, N_SF * S)
    pad = P_SAMP - N_SF * S
    a = _np.pad((_np.float32(1.0) - r1), (0, pad))
    b = _np.pad((r1 * (_np.float32(1.0) - r2)), (0, pad))
    c = _np.pad((r1 * r2), (0, pad))
    m = (_np.arange(P_SAMP) < N_SF * S).astype(_np.float32).reshape(P_SAMP, 1)
    return a, b, c, m


_A_CONST, _B_CONST, _C_CONST, _MASK_CONST = _sample_consts()


def kernel(original_vertices, original_faces, simplified_vertices,
           simplified_faces, face_probabilities):
    f32 = jnp.float32
    ov_flat = jnp.pad(original_vertices.reshape(-1).astype(f32), (0, 30720 - 3 * N_OV))
    of_flat = jnp.pad(original_faces.reshape(-1).astype(jnp.int32), (0, 3 * P_OF - 3 * N_OF))
    sv_flat = jnp.pad(simplified_vertices.reshape(-1).astype(f32), (0, 2048 - 3 * N_SV))
    sf_flat = jnp.pad(simplified_faces.reshape(-1).astype(jnp.int32), (0, 3072 - 3 * N_SF))

    a_flat, b_flat, c_flat = (jnp.asarray(_A_CONST), jnp.asarray(_B_CONST),
                              jnp.asarray(_C_CONST))

    (w_fwd, qt_fwd, qt_rev, w_rev,
     vn_fwd, qn_fwd, qn_rev, vn_rev) = _sc_pack(
        ov_flat, of_flat, sv_flat, sf_flat, a_flat, b_flat, c_flat)

    p_pad = jnp.pad(face_probabilities.astype(f32), (0, P_Q - N_SF)).reshape(P_Q, 1)
    mask = jnp.asarray(_MASK_CONST)

    fwd_term, sum_p = _fwd_tc(qt_fwd, w_fwd, vn_fwd.reshape(1, P_OF),
                              qn_fwd.reshape(P_Q, 1), p_pad)
    rev_sum, rev_max = _rev_tc(qt_rev, w_rev, vn_rev.reshape(1, P_OV),
                               qn_rev.reshape(P_SAMP, 1), mask)

    rev_term = 0.1 * sum_p[0, 0] * rev_sum[0, 0] / rev_max[0, 0]
    return fwd_term[0, 0] + rev_term


# speedup vs baseline: 4.3205x; 1.0252x over previous
import functools

import numpy as _np

import jax
import jax.numpy as jnp
from jax import lax
from jax.experimental import pallas as pl
from jax.experimental.pallas import tpu as pltpu
from jax.experimental.pallas import tpu_sc as plsc

N_OV = 10000
N_OF = 20000
N_SV = 600
N_SF = 1000
S = 8

P_OV = 10240
P_OF = 20480
P_Q = 1024
P_SAMP = 8192

NC = 2
NS = 16
NW = NC * NS
LANES = 16

WF_PER = P_OF // NW
QF_PER = P_Q // NW
QR_PER = P_SAMP // NW
WR_PER = P_OV // NW

BIG = 1e30


def _sc_pack(ov_flat, of_flat, sv_flat, sf_flat, a_flat, b_flat, c_flat):
    mesh = plsc.VectorSubcoreMesh(core_axis_name="c", subcore_axis_name="s")

    @functools.partial(
        pl.kernel,
        mesh=mesh,
        compiler_params=pltpu.CompilerParams(needs_layout_passes=False),
        out_type=[
            jax.ShapeDtypeStruct((8, P_OF), jnp.float32),
            jax.ShapeDtypeStruct((8, P_Q), jnp.float32),
            jax.ShapeDtypeStruct((8, P_SAMP), jnp.float32),
            jax.ShapeDtypeStruct((8, P_OV), jnp.float32),
            jax.ShapeDtypeStruct((P_OF,), jnp.float32),
            jax.ShapeDtypeStruct((P_Q,), jnp.float32),
            jax.ShapeDtypeStruct((P_SAMP,), jnp.float32),
            jax.ShapeDtypeStruct((P_OV,), jnp.float32),
        ],
        scratch_types=[
            pltpu.VMEM((30720,), jnp.float32),
            pltpu.VMEM((3 * WF_PER,), jnp.int32),
            pltpu.VMEM((2048,), jnp.float32),
            pltpu.VMEM((3072,), jnp.int32),
            pltpu.VMEM((QR_PER,), jnp.float32),
            pltpu.VMEM((QR_PER,), jnp.float32),
            pltpu.VMEM((QR_PER,), jnp.float32),
            pltpu.VMEM((8, WF_PER), jnp.float32),
            pltpu.VMEM((8, 128), jnp.float32),
            pltpu.VMEM((8, QR_PER), jnp.float32),
            pltpu.VMEM((8, 128), jnp.float32),
            pltpu.VMEM((WF_PER,), jnp.float32),
            pltpu.VMEM((128,), jnp.float32),
            pltpu.VMEM((QR_PER,), jnp.float32),
            pltpu.VMEM((128,), jnp.float32),
        ],
    )
    def body(ov_h, of_h, sv_h, sf_h, a_h, b_h, c_h,
             wf_h, qf_h, qr_h, wr_h, vnf_h, qnf_h, qnr_h, vnr_h,
             ov_v, of_v, sv_v, sf_v, a_v, b_v, c_v,
             wf_s, qf_s, qr_s, wr_s, vnf_s, qnf_s, qnr_s, vnr_s):
        wid = lax.axis_index("s") * NC + lax.axis_index("c")

        pltpu.sync_copy(ov_h, ov_v)
        pltpu.sync_copy(of_h.at[pl.ds(wid * 3 * WF_PER, 3 * WF_PER)], of_v)
        pltpu.sync_copy(sv_h, sv_v)
        pltpu.sync_copy(sf_h, sf_v)
        pltpu.sync_copy(a_h.at[pl.ds(wid * QR_PER, QR_PER)], a_v)
        pltpu.sync_copy(b_h.at[pl.ds(wid * QR_PER, QR_PER)], b_v)
        pltpu.sync_copy(c_h.at[pl.ds(wid * QR_PER, QR_PER)], c_v)

        iota = lax.iota(jnp.int32, LANES)
        ones = jnp.full((LANES,), 1.0, jnp.float32)
        zeros = jnp.zeros((LANES,), jnp.float32)

        def gat_f(ref, idx):
            return plsc.load_gather(ref, [idx])

        def tri_coords(faces_ref, fidx):
            g0 = plsc.load_gather(faces_ref, [fidx * 3])
            g1 = plsc.load_gather(faces_ref, [fidx * 3 + 1])
            g2 = plsc.load_gather(faces_ref, [fidx * 3 + 2])
            return g0, g1, g2

        def wf_body(i, carry):
            base = i * LANES
            col_l = base + iota
            g0, g1, g2 = tri_coords(of_v, col_l)
            third = jnp.full((LANES,), 1.0 / 3.0, jnp.float32)
            bx = (gat_f(ov_v, g0 * 3) + gat_f(ov_v, g1 * 3) + gat_f(ov_v, g2 * 3)) * third
            by = (gat_f(ov_v, g0 * 3 + 1) + gat_f(ov_v, g1 * 3 + 1) + gat_f(ov_v, g2 * 3 + 1)) * third
            bz = (gat_f(ov_v, g0 * 3 + 2) + gat_f(ov_v, g1 * 3 + 2) + gat_f(ov_v, g2 * 3 + 2)) * third
            col_g = col_l + wid * WF_PER
            nrm = bx * bx + by * by + bz * bz
            nrm = jnp.where(col_g < N_OF, nrm, BIG)
            sl = pl.ds(base, LANES)
            wf_s[0, sl] = -2.0 * bx
            wf_s[1, sl] = -2.0 * by
            wf_s[2, sl] = -2.0 * bz
            wf_s[3, sl] = zeros
            wf_s[4, sl] = zeros
            wf_s[5, sl] = zeros
            wf_s[6, sl] = zeros
            wf_s[7, sl] = zeros
            vnf_s[sl] = nrm
            return carry

        lax.fori_loop(0, WF_PER // LANES, wf_body, 0)

        @pl.when(wid < 8)
        def _qt_fwd():
            def qf_body(i, carry):
                base = i * LANES
                col_g = base + iota + wid * 128
                g0, g1, g2 = tri_coords(sf_v, col_g)
                third = jnp.full((LANES,), 1.0 / 3.0, jnp.float32)
                bx = (gat_f(sv_v, g0 * 3) + gat_f(sv_v, g1 * 3) + gat_f(sv_v, g2 * 3)) * third
                by = (gat_f(sv_v, g0 * 3 + 1) + gat_f(sv_v, g1 * 3 + 1) + gat_f(sv_v, g2 * 3 + 1)) * third
                bz = (gat_f(sv_v, g0 * 3 + 2) + gat_f(sv_v, g1 * 3 + 2) + gat_f(sv_v, g2 * 3 + 2)) * third
                sl = pl.ds(base, LANES)
                qf_s[0, sl] = bx
                qf_s[1, sl] = by
                qf_s[2, sl] = bz
                qf_s[3, sl] = zeros
                qf_s[4, sl] = zeros
                qf_s[5, sl] = zeros
                qf_s[6, sl] = zeros
                qf_s[7, sl] = zeros
                qnf_s[sl] = bx * bx + by * by + bz * bz
                return carry

            lax.fori_loop(0, 128 // LANES, qf_body, 0)
            pltpu.sync_copy(qf_s, qf_h.at[:, pl.ds(wid * 128, 128)])
            pltpu.sync_copy(qnf_s, qnf_h.at[pl.ds(wid * 128, 128)])

        def qr_body(i, carry):
            base = i * LANES
            col_l = base + iota
            col_g = col_l + wid * QR_PER
            f = lax.shift_right_logical(col_g, 3)
            g0, g1, g2 = tri_coords(sf_v, f)
            sl = pl.ds(base, LANES)
            av = a_v[sl]
            bv = b_v[sl]
            cv = c_v[sl]
            sx = av * gat_f(sv_v, g0 * 3) + bv * gat_f(sv_v, g1 * 3) + cv * gat_f(sv_v, g2 * 3)
            sy = av * gat_f(sv_v, g0 * 3 + 1) + bv * gat_f(sv_v, g1 * 3 + 1) + cv * gat_f(sv_v, g2 * 3 + 1)
            sz = av * gat_f(sv_v, g0 * 3 + 2) + bv * gat_f(sv_v, g1 * 3 + 2) + cv * gat_f(sv_v, g2 * 3 + 2)
            qr_s[0, sl] = sx
            qr_s[1, sl] = sy
            qr_s[2, sl] = sz
            qr_s[3, sl] = zeros
            qr_s[4, sl] = zeros
            qr_s[5, sl] = zeros
            qr_s[6, sl] = zeros
            qr_s[7, sl] = zeros
            qnr_s[sl] = sx * sx + sy * sy + sz * sz
            return carry

        lax.fori_loop(0, QR_PER // LANES, qr_body, 0)

        for k in range(3):
            chunk = wid + 32 * k

            @pl.when(chunk < 80)
            def _wr_chunk(chunk=chunk):
                def wr_body(i, carry):
                    base = i * LANES
                    col_g = base + iota + chunk * 128
                    vx = gat_f(ov_v, col_g * 3)
                    vy = gat_f(ov_v, col_g * 3 + 1)
                    vz = gat_f(ov_v, col_g * 3 + 2)
                    nrm = vx * vx + vy * vy + vz * vz
                    nrm = jnp.where(col_g < N_OV, nrm, BIG)
                    sl = pl.ds(base, LANES)
                    wr_s[0, sl] = -2.0 * vx
                    wr_s[1, sl] = -2.0 * vy
                    wr_s[2, sl] = -2.0 * vz
                    wr_s[3, sl] = zeros
                    wr_s[4, sl] = zeros
                    wr_s[5, sl] = zeros
                    wr_s[6, sl] = zeros
                    wr_s[7, sl] = zeros
                    vnr_s[sl] = nrm
                    return carry

                lax.fori_loop(0, 128 // LANES, wr_body, 0)
                pltpu.sync_copy(wr_s, wr_h.at[:, pl.ds(chunk * 128, 128)])
                pltpu.sync_copy(vnr_s, vnr_h.at[pl.ds(chunk * 128, 128)])

        pltpu.sync_copy(wf_s, wf_h.at[:, pl.ds(wid * WF_PER, WF_PER)])
        pltpu.sync_copy(qr_s, qr_h.at[:, pl.ds(wid * QR_PER, QR_PER)])
        pltpu.sync_copy(vnf_s, vnf_h.at[pl.ds(wid * WF_PER, WF_PER)])
        pltpu.sync_copy(qnr_s, qnr_h.at[pl.ds(wid * QR_PER, QR_PER)])

    return body(ov_flat, of_flat, sv_flat, sf_flat, a_flat, b_flat, c_flat)


def _stack3_in_kernel(x):
    xh = x.astype(jnp.bfloat16)
    xl = (x - xh.astype(jnp.float32)).astype(jnp.bfloat16)
    return xh, xl


def _fwd_tc(qt, w, vn, qn, p_pad):
    jblk = 4096
    njb = P_OF // jblk
    dn = (((0,), (0,)), ((), ()))

    def body(qt_ref, w_ref, vn_ref, qn_ref, p_ref, fwd_out, sump_out, acc):
        j = pl.program_id(0)
        qh, ql = _stack3_in_kernel(qt_ref[...])
        wh, wl = _stack3_in_kernel(w_ref[...])
        q3 = jnp.concatenate([qh, qh, ql], axis=0)
        w3 = jnp.concatenate([wh, wl, wh], axis=0)
        d = lax.dot_general(q3, w3, dn, preferred_element_type=jnp.float32)
        vnr = vn_ref[...]
        m = None
        for k in range(jblk // 128):
            sl = slice(k * 128, (k + 1) * 128)
            chunk = d[:, sl] + vnr[:, sl]
            m = chunk if m is None else jnp.minimum(m, chunk)
        acc[...] = jnp.where(j == 0, m, jnp.minimum(acc[...], m))

        @pl.when(j == njb - 1)
        def _():
            p = p_ref[...]
            sp = jnp.sum(p)
            row_min = jnp.min(acc[...], axis=1, keepdims=True) + qn_ref[...]
            fwd_out[...] = jnp.full((1, 1), jnp.sum(p * row_min) + 1e-4 * (float(N_SF) - sp), jnp.float32)
            sump_out[...] = jnp.full((1, 1), sp, jnp.float32)

    return pl.pallas_call(
        body,
        grid=(njb,),
        in_specs=[
            pl.BlockSpec((8, P_Q), lambda j: (0, 0)),
            pl.BlockSpec((8, jblk), lambda j: (0, j)),
            pl.BlockSpec((1, jblk), lambda j: (0, j)),
            pl.BlockSpec((P_Q, 1), lambda j: (0, 0)),
            pl.BlockSpec((P_Q, 1), lambda j: (0, 0)),
        ],
        out_specs=[
            pl.BlockSpec((1, 1), lambda j: (0, 0)),
            pl.BlockSpec((1, 1), lambda j: (0, 0)),
        ],
        out_shape=[
            jax.ShapeDtypeStruct((1, 1), jnp.float32),
            jax.ShapeDtypeStruct((1, 1), jnp.float32),
        ],
        scratch_shapes=[pltpu.VMEM((P_Q, 128), jnp.float32)],
    )(qt, w, vn, qn, p_pad)


def _rev_tc(qt, w, vn, qn, mask):
    iblk = 1024
    jblk = 2560
    nib = P_SAMP // iblk
    njb = P_OV // jblk
    dn = (((0,), (0,)), ((), ()))

    def body(qt_ref, w_ref, vn_ref, qn_ref, mask_ref, sum_out, max_out, acc, ssum, smax):
        i = pl.program_id(0)
        j = pl.program_id(1)
        qh, ql = _stack3_in_kernel(qt_ref[...])
        wh, wl = _stack3_in_kernel(w_ref[...])
        q3 = jnp.concatenate([qh, qh, ql], axis=0)
        w3 = jnp.concatenate([wh, wl, wh], axis=0)
        d = lax.dot_general(q3, w3, dn, preferred_element_type=jnp.float32)
        vnr = vn_ref[...]
        m = None
        for k in range(jblk // 128):
            sl = slice(k * 128, (k + 1) * 128)
            chunk = d[:, sl] + vnr[:, sl]
            m = chunk if m is None else jnp.minimum(m, chunk)
        acc[...] = jnp.where(j == 0, m, jnp.minimum(acc[...], m))

        @pl.when(j == njb - 1)
        def _():
            row_min = jnp.min(acc[...], axis=1, keepdims=True) + qn_ref[...]
            mm = mask_ref[...] * row_min
            s = jnp.sum(mm)
            mx = jnp.max(mm)
            ssum[0] = jnp.where(i == 0, s, ssum[0] + s)
            smax[0] = jnp.where(i == 0, mx, jnp.maximum(smax[0], mx))
            sum_out[...] = jnp.full((1, 1), ssum[0], jnp.float32)
            max_out[...] = jnp.full((1, 1), smax[0], jnp.float32)

    return pl.pallas_call(
        body,
        grid=(nib, njb),
        in_specs=[
            pl.BlockSpec((8, iblk), lambda i, j: (0, i)),
            pl.BlockSpec((8, jblk), lambda i, j: (0, j)),
            pl.BlockSpec((1, jblk), lambda i, j: (0, j)),
            pl.BlockSpec((iblk, 1), lambda i, j: (i, 0)),
            pl.BlockSpec((iblk, 1), lambda i, j: (i, 0)),
        ],
        out_specs=[
            pl.BlockSpec((1, 1), lambda i, j: (0, 0)),
            pl.BlockSpec((1, 1), lambda i, j: (0, 0)),
        ],
        out_shape=[
            jax.ShapeDtypeStruct((1, 1), jnp.float32),
            jax.ShapeDtypeStruct((1, 1), jnp.float32),
        ],
        scratch_shapes=[
            pltpu.VMEM((iblk, 128), jnp.float32),
            pltpu.SMEM((1,), jnp.float32),
            pltpu.SMEM((1,), jnp.float32),
        ],
    )(qt, w, vn, qn, mask)


def _tf2x32(k1, k2, x0, x1):
    def rotl(x, d):
        return ((x << _np.uint32(d)) | (x >> _np.uint32(32 - d))).astype(_np.uint32)
    rot = [(13, 15, 26, 6), (17, 29, 16, 24)]
    ks = [_np.uint32(k1), _np.uint32(k2),
          _np.uint32(k1 ^ k2 ^ _np.uint32(0x1BD11BDA))]
    x = [(x0 + ks[0]).astype(_np.uint32), (x1 + ks[1]).astype(_np.uint32)]
    for i, (rs, ka, kb) in enumerate([(rot[0], 1, 2), (rot[1], 2, 0),
                                      (rot[0], 0, 1), (rot[1], 1, 2), (rot[0], 2, 0)]):
        for r in rs:
            x[0] = (x[0] + x[1]).astype(_np.uint32)
            x[1] = rotl(x[1], r)
            x[1] = (x[1] ^ x[0]).astype(_np.uint32)
        x[0] = (x[0] + ks[ka]).astype(_np.uint32)
        x[1] = (x[1] + ks[kb] + _np.uint32(i + 1)).astype(_np.uint32)
    return x[0], x[1]


def _tf_uniform(key, n):
    b1, b2 = _tf2x32(key[0], key[1], _np.zeros(n, _np.uint32),
                     _np.arange(n, dtype=_np.uint32))
    bits = b1 ^ b2
    flt = ((bits >> _np.uint32(9)) | _np.uint32(0x3F800000)).view(_np.float32)
    return _np.maximum(_np.float32(0.0), flt - _np.float32(1.0))


def _sample_consts():
    key = _np.array([0, 42], _np.uint32)
    b1, b2 = _tf2x32(key[0], key[1], _np.zeros(2, _np.uint32),
                     _np.arange(2, dtype=_np.uint32))
    k1, k2 = _np.stack([b1, b2], axis=1)
    r1 = _np.sqrt(_tf_uniform(k1, N_SF * S))
    r2 = _tf_uniform(k2, N_SF * S)
    pad = P_SAMP - N_SF * S
    a = _np.pad((_np.float32(1.0) - r1), (0, pad))
    b = _np.pad((r1 * (_np.float32(1.0) - r2)), (0, pad))
    c = _np.pad((r1 * r2), (0, pad))
    m = (_np.arange(P_SAMP) < N_SF * S).astype(_np.float32).reshape(P_SAMP, 1)
    return a, b, c, m


_A_CONST, _B_CONST, _C_CONST, _MASK_CONST = _sample_consts()


def kernel(original_vertices, original_faces, simplified_vertices,
           simplified_faces, face_probabilities):
    f32 = jnp.float32
    ov_flat = jnp.pad(original_vertices.reshape(-1).astype(f32), (0, 30720 - 3 * N_OV))
    of_flat = jnp.pad(original_faces.reshape(-1).astype(jnp.int32), (0, 3 * P_OF - 3 * N_OF))
    sv_flat = jnp.pad(simplified_vertices.reshape(-1).astype(f32), (0, 2048 - 3 * N_SV))
    sf_flat = jnp.pad(simplified_faces.reshape(-1).astype(jnp.int32), (0, 3072 - 3 * N_SF))

    a_flat, b_flat, c_flat = (jnp.asarray(_A_CONST), jnp.asarray(_B_CONST),
                              jnp.asarray(_C_CONST))

    (w_fwd, qt_fwd, qt_rev, w_rev,
     vn_fwd, qn_fwd, qn_rev, vn_rev) = _sc_pack(
        ov_flat, of_flat, sv_flat, sf_flat, a_flat, b_flat, c_flat)

    p_pad = jnp.pad(face_probabilities.astype(f32), (0, P_Q - N_SF)).reshape(P_Q, 1)
    mask = jnp.asarray(_MASK_CONST)

    fwd_term, sum_p = _fwd_tc(qt_fwd, w_fwd, vn_fwd.reshape(1, P_OF),
                              qn_fwd.reshape(P_Q, 1), p_pad)
    rev_sum, rev_max = _rev_tc(qt_rev, w_rev, vn_rev.reshape(1, P_OV),
                               qn_rev.reshape(P_SAMP, 1), mask)

    rev_term = 0.1 * sum_p[0, 0] * rev_sum[0, 0] / rev_max[0, 0]
    return fwd_term[0, 0] + rev_term


# speedup vs baseline: 4.3656x; 1.0104x over previous
import functools

import numpy as _np

import jax
import jax.numpy as jnp
from jax import lax
from jax.experimental import pallas as pl
from jax.experimental.pallas import tpu as pltpu
from jax.experimental.pallas import tpu_sc as plsc

N_OV = 10000
N_OF = 20000
N_SV = 600
N_SF = 1000
S = 8

P_OV = 10240
P_OF = 20480
P_Q = 1024
P_SAMP = 8192

NC = 2
NS = 16
NW = NC * NS
LANES = 16

WF_PER = P_OF // NW
QF_PER = P_Q // NW
QR_PER = P_SAMP // NW
WR_PER = P_OV // NW

BIG = 1e30


def _sc_pack(ov_flat, of_flat, sv_flat, sf_flat, a_flat, b_flat, c_flat):
    mesh = plsc.VectorSubcoreMesh(core_axis_name="c", subcore_axis_name="s")

    @functools.partial(
        pl.kernel,
        mesh=mesh,
        compiler_params=pltpu.CompilerParams(needs_layout_passes=False),
        out_type=[
            jax.ShapeDtypeStruct((8, P_OF), jnp.float32),
            jax.ShapeDtypeStruct((8, P_Q), jnp.float32),
            jax.ShapeDtypeStruct((8, P_SAMP), jnp.float32),
            jax.ShapeDtypeStruct((8, P_OV), jnp.float32),
            jax.ShapeDtypeStruct((P_OF,), jnp.float32),
            jax.ShapeDtypeStruct((P_Q,), jnp.float32),
            jax.ShapeDtypeStruct((P_SAMP,), jnp.float32),
            jax.ShapeDtypeStruct((P_OV,), jnp.float32),
        ],
        scratch_types=[
            pltpu.VMEM((30720,), jnp.float32),
            pltpu.VMEM((3 * WF_PER,), jnp.int32),
            pltpu.VMEM((2048,), jnp.float32),
            pltpu.VMEM((3072,), jnp.int32),
            pltpu.VMEM((QR_PER,), jnp.float32),
            pltpu.VMEM((QR_PER,), jnp.float32),
            pltpu.VMEM((QR_PER,), jnp.float32),
            pltpu.VMEM((8, WF_PER), jnp.float32),
            pltpu.VMEM((8, 128), jnp.float32),
            pltpu.VMEM((8, QR_PER), jnp.float32),
            pltpu.VMEM((8, 128), jnp.float32),
            pltpu.VMEM((WF_PER,), jnp.float32),
            pltpu.VMEM((128,), jnp.float32),
            pltpu.VMEM((QR_PER,), jnp.float32),
            pltpu.VMEM((128,), jnp.float32),
        ],
    )
    def body(ov_h, of_h, sv_h, sf_h, a_h, b_h, c_h,
             wf_h, qf_h, qr_h, wr_h, vnf_h, qnf_h, qnr_h, vnr_h,
             ov_v, of_v, sv_v, sf_v, a_v, b_v, c_v,
             wf_s, qf_s, qr_s, wr_s, vnf_s, qnf_s, qnr_s, vnr_s):
        wid = lax.axis_index("s") * NC + lax.axis_index("c")

        pltpu.sync_copy(ov_h, ov_v)
        pltpu.sync_copy(of_h.at[pl.ds(wid * 3 * WF_PER, 3 * WF_PER)], of_v)
        pltpu.sync_copy(sv_h, sv_v)
        pltpu.sync_copy(sf_h, sf_v)
        pltpu.sync_copy(a_h.at[pl.ds(wid * QR_PER, QR_PER)], a_v)
        pltpu.sync_copy(b_h.at[pl.ds(wid * QR_PER, QR_PER)], b_v)
        pltpu.sync_copy(c_h.at[pl.ds(wid * QR_PER, QR_PER)], c_v)

        iota = lax.iota(jnp.int32, LANES)
        ones = jnp.full((LANES,), 1.0, jnp.float32)
        zeros = jnp.zeros((LANES,), jnp.float32)

        def gat_f(ref, idx):
            return plsc.load_gather(ref, [idx])

        def tri_coords(faces_ref, fidx):
            g0 = plsc.load_gather(faces_ref, [fidx * 3])
            g1 = plsc.load_gather(faces_ref, [fidx * 3 + 1])
            g2 = plsc.load_gather(faces_ref, [fidx * 3 + 2])
            return g0, g1, g2

        def wf_body(i, carry):
            base = i * LANES
            col_l = base + iota
            g0, g1, g2 = tri_coords(of_v, col_l)
            third = jnp.full((LANES,), 1.0 / 3.0, jnp.float32)
            bx = (gat_f(ov_v, g0 * 3) + gat_f(ov_v, g1 * 3) + gat_f(ov_v, g2 * 3)) * third
            by = (gat_f(ov_v, g0 * 3 + 1) + gat_f(ov_v, g1 * 3 + 1) + gat_f(ov_v, g2 * 3 + 1)) * third
            bz = (gat_f(ov_v, g0 * 3 + 2) + gat_f(ov_v, g1 * 3 + 2) + gat_f(ov_v, g2 * 3 + 2)) * third
            col_g = col_l + wid * WF_PER
            nrm = bx * bx + by * by + bz * bz
            nrm = jnp.where(col_g < N_OF, nrm, BIG)
            sl = pl.ds(base, LANES)
            wf_s[0, sl] = -2.0 * bx
            wf_s[1, sl] = -2.0 * by
            wf_s[2, sl] = -2.0 * bz
            wf_s[3, sl] = zeros
            wf_s[4, sl] = zeros
            wf_s[5, sl] = zeros
            wf_s[6, sl] = zeros
            wf_s[7, sl] = zeros
            vnf_s[sl] = nrm
            return carry

        lax.fori_loop(0, WF_PER // LANES, wf_body, 0)

        @pl.when(wid < 8)
        def _qt_fwd():
            def qf_body(i, carry):
                base = i * LANES
                col_g = base + iota + wid * 128
                g0, g1, g2 = tri_coords(sf_v, col_g)
                third = jnp.full((LANES,), 1.0 / 3.0, jnp.float32)
                bx = (gat_f(sv_v, g0 * 3) + gat_f(sv_v, g1 * 3) + gat_f(sv_v, g2 * 3)) * third
                by = (gat_f(sv_v, g0 * 3 + 1) + gat_f(sv_v, g1 * 3 + 1) + gat_f(sv_v, g2 * 3 + 1)) * third
                bz = (gat_f(sv_v, g0 * 3 + 2) + gat_f(sv_v, g1 * 3 + 2) + gat_f(sv_v, g2 * 3 + 2)) * third
                sl = pl.ds(base, LANES)
                qf_s[0, sl] = bx
                qf_s[1, sl] = by
                qf_s[2, sl] = bz
                qf_s[3, sl] = zeros
                qf_s[4, sl] = zeros
                qf_s[5, sl] = zeros
                qf_s[6, sl] = zeros
                qf_s[7, sl] = zeros
                qnf_s[sl] = bx * bx + by * by + bz * bz
                return carry

            lax.fori_loop(0, 128 // LANES, qf_body, 0)
            pltpu.sync_copy(qf_s, qf_h.at[:, pl.ds(wid * 128, 128)])
            pltpu.sync_copy(qnf_s, qnf_h.at[pl.ds(wid * 128, 128)])

        def qr_body(i, carry):
            base = i * LANES
            col_l = base + iota
            col_g = col_l + wid * QR_PER
            f = lax.shift_right_logical(col_g, 3)
            g0, g1, g2 = tri_coords(sf_v, f)
            sl = pl.ds(base, LANES)
            av = a_v[sl]
            bv = b_v[sl]
            cv = c_v[sl]
            sx = av * gat_f(sv_v, g0 * 3) + bv * gat_f(sv_v, g1 * 3) + cv * gat_f(sv_v, g2 * 3)
            sy = av * gat_f(sv_v, g0 * 3 + 1) + bv * gat_f(sv_v, g1 * 3 + 1) + cv * gat_f(sv_v, g2 * 3 + 1)
            sz = av * gat_f(sv_v, g0 * 3 + 2) + bv * gat_f(sv_v, g1 * 3 + 2) + cv * gat_f(sv_v, g2 * 3 + 2)
            qr_s[0, sl] = sx
            qr_s[1, sl] = sy
            qr_s[2, sl] = sz
            qr_s[3, sl] = zeros
            qr_s[4, sl] = zeros
            qr_s[5, sl] = zeros
            qr_s[6, sl] = zeros
            qr_s[7, sl] = zeros
            qnr_s[sl] = sx * sx + sy * sy + sz * sz
            return carry

        lax.fori_loop(0, QR_PER // LANES, qr_body, 0)

        for k in range(3):
            chunk = wid + 32 * k

            @pl.when(chunk < 80)
            def _wr_chunk(chunk=chunk):
                def wr_body(i, carry):
                    base = i * LANES
                    col_g = base + iota + chunk * 128
                    vx = gat_f(ov_v, col_g * 3)
                    vy = gat_f(ov_v, col_g * 3 + 1)
                    vz = gat_f(ov_v, col_g * 3 + 2)
                    nrm = vx * vx + vy * vy + vz * vz
                    nrm = jnp.where(col_g < N_OV, nrm, BIG)
                    sl = pl.ds(base, LANES)
                    wr_s[0, sl] = -2.0 * vx
                    wr_s[1, sl] = -2.0 * vy
                    wr_s[2, sl] = -2.0 * vz
                    wr_s[3, sl] = zeros
                    wr_s[4, sl] = zeros
                    wr_s[5, sl] = zeros
                    wr_s[6, sl] = zeros
                    wr_s[7, sl] = zeros
                    vnr_s[sl] = nrm
                    return carry

                lax.fori_loop(0, 128 // LANES, wr_body, 0)
                pltpu.sync_copy(wr_s, wr_h.at[:, pl.ds(chunk * 128, 128)])
                pltpu.sync_copy(vnr_s, vnr_h.at[pl.ds(chunk * 128, 128)])

        pltpu.sync_copy(wf_s, wf_h.at[:, pl.ds(wid * WF_PER, WF_PER)])
        pltpu.sync_copy(qr_s, qr_h.at[:, pl.ds(wid * QR_PER, QR_PER)])
        pltpu.sync_copy(vnf_s, vnf_h.at[pl.ds(wid * WF_PER, WF_PER)])
        pltpu.sync_copy(qnr_s, qnr_h.at[pl.ds(wid * QR_PER, QR_PER)])

    return body(ov_flat, of_flat, sv_flat, sf_flat, a_flat, b_flat, c_flat)


def _stack3_in_kernel(x):
    xh = x.astype(jnp.bfloat16)
    xl = (x - xh.astype(jnp.float32)).astype(jnp.bfloat16)
    return xh, xl


def _fwd_tc(qt, w, vn, qn, p_pad):
    jblk = 4096
    njb = P_OF // jblk
    dn = (((0,), (0,)), ((), ()))

    def body(qt_ref, w_ref, vn_ref, qn_ref, p_ref, fwd_out, sump_out, acc):
        j = pl.program_id(0)
        qh, ql = _stack3_in_kernel(qt_ref[...])
        wh, wl = _stack3_in_kernel(w_ref[...])
        q3 = jnp.concatenate([qh, qh, ql], axis=0)
        w3 = jnp.concatenate([wh, wl, wh], axis=0)
        d = lax.dot_general(q3, w3, dn, preferred_element_type=jnp.float32)
        vnr = vn_ref[...]
        m = None
        for k in range(jblk // 128):
            sl = slice(k * 128, (k + 1) * 128)
            chunk = d[:, sl] + vnr[:, sl]
            m = chunk if m is None else jnp.minimum(m, chunk)
        acc[...] = jnp.where(j == 0, m, jnp.minimum(acc[...], m))

        @pl.when(j == njb - 1)
        def _():
            p = p_ref[...]
            sp = jnp.sum(p)
            row_min = jnp.min(acc[...], axis=1, keepdims=True) + qn_ref[...]
            fwd_out[...] = jnp.full((1, 1), jnp.sum(p * row_min) + 1e-4 * (float(N_SF) - sp), jnp.float32)
            sump_out[...] = jnp.full((1, 1), sp, jnp.float32)

    return pl.pallas_call(
        body,
        grid=(njb,),
        in_specs=[
            pl.BlockSpec((8, P_Q), lambda j: (0, 0)),
            pl.BlockSpec((8, jblk), lambda j: (0, j)),
            pl.BlockSpec((1, jblk), lambda j: (0, j)),
            pl.BlockSpec((P_Q, 1), lambda j: (0, 0)),
            pl.BlockSpec((P_Q, 1), lambda j: (0, 0)),
        ],
        out_specs=[
            pl.BlockSpec((1, 1), lambda j: (0, 0)),
            pl.BlockSpec((1, 1), lambda j: (0, 0)),
        ],
        out_shape=[
            jax.ShapeDtypeStruct((1, 1), jnp.float32),
            jax.ShapeDtypeStruct((1, 1), jnp.float32),
        ],
        scratch_shapes=[pltpu.VMEM((P_Q, 128), jnp.float32)],
    )(qt, w, vn, qn, p_pad)


def _rev_tc(qt, w, vn, qn, mask, fwd_term, sum_p):
    iblk = 1024
    jblk = 2560
    nib = P_SAMP // iblk
    njb = P_OV // jblk
    dn = (((0,), (0,)), ((), ()))

    def body(qt_ref, w_ref, vn_ref, qn_ref, mask_ref, fwd_ref, sp_ref, loss_out, acc, ssum, smax):
        i = pl.program_id(0)
        j = pl.program_id(1)
        qh, ql = _stack3_in_kernel(qt_ref[...])
        wh, wl = _stack3_in_kernel(w_ref[...])
        q3 = jnp.concatenate([qh, qh, ql], axis=0)
        w3 = jnp.concatenate([wh, wl, wh], axis=0)
        d = lax.dot_general(q3, w3, dn, preferred_element_type=jnp.float32)
        vnr = vn_ref[...]
        m = None
        for k in range(jblk // 128):
            sl = slice(k * 128, (k + 1) * 128)
            chunk = d[:, sl] + vnr[:, sl]
            m = chunk if m is None else jnp.minimum(m, chunk)
        acc[...] = jnp.where(j == 0, m, jnp.minimum(acc[...], m))

        @pl.when(j == njb - 1)
        def _():
            row_min = jnp.min(acc[...], axis=1, keepdims=True) + qn_ref[...]
            mm = mask_ref[...] * row_min
            s = jnp.sum(mm)
            mx = jnp.max(mm)
            ssum[0] = jnp.where(i == 0, s, ssum[0] + s)
            smax[0] = jnp.where(i == 0, mx, jnp.maximum(smax[0], mx))

            @pl.when(i == nib - 1)
            def __():
                rev_term = 0.1 * sp_ref[0, 0] * ssum[0] / smax[0]
                loss_out[...] = jnp.full((1, 1), fwd_ref[0, 0] + rev_term, jnp.float32)

    return pl.pallas_call(
        body,
        grid=(nib, njb),
        in_specs=[
            pl.BlockSpec((8, iblk), lambda i, j: (0, i)),
            pl.BlockSpec((8, jblk), lambda i, j: (0, j)),
            pl.BlockSpec((1, jblk), lambda i, j: (0, j)),
            pl.BlockSpec((iblk, 1), lambda i, j: (i, 0)),
            pl.BlockSpec((iblk, 1), lambda i, j: (i, 0)),
            pl.BlockSpec((1, 1), lambda i, j: (0, 0)),
            pl.BlockSpec((1, 1), lambda i, j: (0, 0)),
        ],
        out_specs=pl.BlockSpec((1, 1), lambda i, j: (0, 0)),
        out_shape=jax.ShapeDtypeStruct((1, 1), jnp.float32),
        scratch_shapes=[
            pltpu.VMEM((iblk, 128), jnp.float32),
            pltpu.SMEM((1,), jnp.float32),
            pltpu.SMEM((1,), jnp.float32),
        ],
    )(qt, w, vn, qn, mask, fwd_term, sum_p)


def _tf2x32(k1, k2, x0, x1):
    def rotl(x, d):
        return ((x << _np.uint32(d)) | (x >> _np.uint32(32 - d))).astype(_np.uint32)
    rot = [(13, 15, 26, 6), (17, 29, 16, 24)]
    ks = [_np.uint32(k1), _np.uint32(k2),
          _np.uint32(k1 ^ k2 ^ _np.uint32(0x1BD11BDA))]
    x = [(x0 + ks[0]).astype(_np.uint32), (x1 + ks[1]).astype(_np.uint32)]
    for i, (rs, ka, kb) in enumerate([(rot[0], 1, 2), (rot[1], 2, 0),
                                      (rot[0], 0, 1), (rot[1], 1, 2), (rot[0], 2, 0)]):
        for r in rs:
            x[0] = (x[0] + x[1]).astype(_np.uint32)
            x[1] = rotl(x[1], r)
            x[1] = (x[1] ^ x[0]).astype(_np.uint32)
        x[0] = (x[0] + ks[ka]).astype(_np.uint32)
        x[1] = (x[1] + ks[kb] + _np.uint32(i + 1)).astype(_np.uint32)
    return x[0], x[1]


def _tf_uniform(key, n):
    b1, b2 = _tf2x32(key[0], key[1], _np.zeros(n, _np.uint32),
                     _np.arange(n, dtype=_np.uint32))
    bits = b1 ^ b2
    flt = ((bits >> _np.uint32(9)) | _np.uint32(0x3F800000)).view(_np.float32)
    return _np.maximum(_np.float32(0.0), flt - _np.float32(1.0))


def _sample_consts():
    key = _np.array([0, 42], _np.uint32)
    b1, b2 = _tf2x32(key[0], key[1], _np.zeros(2, _np.uint32),
                     _np.arange(2, dtype=_np.uint32))
    k1, k2 = _np.stack([b1, b2], axis=1)
    r1 = _np.sqrt(_tf_uniform(k1, N_SF * S))
    r2 = _tf_uniform(k2, N_SF * S)
    pad = P_SAMP - N_SF * S
    a = _np.pad((_np.float32(1.0) - r1), (0, pad))
    b = _np.pad((r1 * (_np.float32(1.0) - r2)), (0, pad))
    c = _np.pad((r1 * r2), (0, pad))
    m = (_np.arange(P_SAMP) < N_SF * S).astype(_np.float32).reshape(P_SAMP, 1)
    return a, b, c, m


_A_CONST, _B_CONST, _C_CONST, _MASK_CONST = _sample_consts()


def kernel(original_vertices, original_faces, simplified_vertices,
           simplified_faces, face_probabilities):
    f32 = jnp.float32
    ov_flat = jnp.pad(original_vertices.reshape(-1).astype(f32), (0, 30720 - 3 * N_OV))
    of_flat = jnp.pad(original_faces.reshape(-1).astype(jnp.int32), (0, 3 * P_OF - 3 * N_OF))
    sv_flat = jnp.pad(simplified_vertices.reshape(-1).astype(f32), (0, 2048 - 3 * N_SV))
    sf_flat = jnp.pad(simplified_faces.reshape(-1).astype(jnp.int32), (0, 3072 - 3 * N_SF))

    a_flat, b_flat, c_flat = (jnp.asarray(_A_CONST), jnp.asarray(_B_CONST),
                              jnp.asarray(_C_CONST))

    (w_fwd, qt_fwd, qt_rev, w_rev,
     vn_fwd, qn_fwd, qn_rev, vn_rev) = _sc_pack(
        ov_flat, of_flat, sv_flat, sf_flat, a_flat, b_flat, c_flat)

    p_pad = jnp.pad(face_probabilities.astype(f32), (0, P_Q - N_SF)).reshape(P_Q, 1)
    mask = jnp.asarray(_MASK_CONST)

    fwd_term, sum_p = _fwd_tc(qt_fwd, w_fwd, vn_fwd.reshape(1, P_OF),
                              qn_fwd.reshape(P_Q, 1), p_pad)
    loss = _rev_tc(qt_rev, w_rev, vn_rev.reshape(1, P_OV),
                   qn_rev.reshape(P_SAMP, 1), mask, fwd_term, sum_p)
    return loss[0, 0]
